# baseline (speedup 1.0000x reference)
import functools

import jax
import jax.numpy as jnp
from jax import lax
from jax.experimental import pallas as pl
from jax.experimental.pallas import tpu as pltpu
from jax.experimental.pallas import tpu_sc as plsc

N = 10000
D = 128
NC = 2
NS = 16
NW = NC * NS
EB = 112
NPAD = 10240
RPT = NPAD // NS

_MESH = dict(core_axis_name="c", subcore_axis_name="s")


def _my_batches(c, s, nbw0, nbw1):
  start = jnp.where(c == 0, s * nbw0, NS * nbw0 + s * nbw1)
  nb = jnp.where(c == 0, nbw0, nbw1)
  return start, nb


def _row_chunks():
  chunks = []
  off = 0
  while off < RPT:
    ln = min(EB, RPT - off)
    chunks.append((off, ln))
    off += ln
  return chunks


def _zero_rows(rows):
  def zb(i, _):
    for v in range(D // 16):
      rows[i, pl.ds(v * 16, 16)] = jnp.zeros((16,), jnp.float32)
    return 0
  lax.fori_loop(0, EB, zb, 0)


def _zero_vec(vec, n):
  def zb(i, _):
    vec[pl.ds(i * 16, 16)] = jnp.zeros((16,), jnp.float32)
    return 0
  lax.fori_loop(0, n // 16, zb, 0)


def _sc_degree(nbw0, nbw1):
  @functools.partial(
      pl.kernel,
      out_type=jax.ShapeDtypeStruct((NC, 2, NPAD), jnp.float32),
      mesh=plsc.VectorSubcoreMesh(**_MESH),
      scratch_types=[
          pltpu.VMEM_SHARED((NPAD,), jnp.float32),
          pltpu.VMEM_SHARED((NPAD,), jnp.float32),
          pltpu.VMEM((4, EB), jnp.int32),
          pltpu.VMEM((4, EB), jnp.int32),
          pltpu.VMEM((EB,), jnp.float32),
          pltpu.VMEM((RPT,), jnp.float32),
          pltpu.SemaphoreType.DMA,
      ],
  )
  def deg_kernel(srcr, dstr, out, acc_s, acc_d, sv, dv, ones, stage, isem):
    c = lax.axis_index("c")
    s = lax.axis_index("s")
    start, nb = _my_batches(c, s, nbw0, nbw1)
    base = s * RPT
    _zero_vec(stage, RPT)
    def ob(i, _):
      ones[pl.ds(i * 16, 16)] = jnp.ones((16,), jnp.float32)
      return 0
    lax.fori_loop(0, EB // 16, ob, 0)
    pltpu.sync_copy(stage, acc_s.at[pl.ds(base, RPT)])
    pltpu.sync_copy(stage, acc_d.at[pl.ds(base, RPT)])
    plsc.subcore_barrier()
    pltpu.sync_copy(srcr.at[start], sv.at[0])
    pltpu.sync_copy(dstr.at[start], dv.at[0])
    pltpu.async_copy(srcr.at[start + 1], sv.at[1], isem)
    pltpu.async_copy(dstr.at[start + 1], dv.at[1], isem)
    def eb(b, _):
      i0 = lax.rem(b, 4)
      i1 = lax.rem(b + 1, 4)
      pltpu.sync_copy(ones, acc_s.at[sv.at[i0]], add=True)
      pltpu.sync_copy(ones, acc_d.at[dv.at[i0]], add=True)
      @pl.when(b + 1 < nb)
      def _():
        pltpu.make_async_copy(srcr.at[start + b + 1], sv.at[i1], isem).wait()
        pltpu.make_async_copy(dstr.at[start + b + 1], dv.at[i1], isem).wait()
      @pl.when(b + 2 < nb)
      def _():
        i2 = lax.rem(b + 2, 4)
        pltpu.async_copy(srcr.at[start + b + 2], sv.at[i2], isem)
        pltpu.async_copy(dstr.at[start + b + 2], dv.at[i2], isem)
      return 0
    lax.fori_loop(0, nb, eb, 0)
    plsc.subcore_barrier()
    pltpu.sync_copy(acc_s.at[pl.ds(base, RPT)], stage)
    pltpu.sync_copy(stage, out.at[c, 0, pl.ds(base, RPT)])
    pltpu.sync_copy(acc_d.at[pl.ds(base, RPT)], stage)
    pltpu.sync_copy(stage, out.at[c, 1, pl.ds(base, RPT)])

  return deg_kernel


def _sc_prop(nbw0, nbw1):
  @functools.partial(
      pl.kernel,
      out_type=jax.ShapeDtypeStruct((NC, NPAD, D), jnp.float32),
      mesh=plsc.VectorSubcoreMesh(**_MESH),
      scratch_types=[
          pltpu.VMEM_SHARED((NPAD, D), jnp.float32),
          pltpu.VMEM((4, EB), jnp.int32),
          pltpu.VMEM((4, EB), jnp.int32),
          pltpu.VMEM((3, EB, D), jnp.float32),
          pltpu.SemaphoreType.DMA,
          pltpu.SemaphoreType.DMA,
          pltpu.SemaphoreType.DMA,
          pltpu.SemaphoreType.DMA,
      ],
  )
  def prop_kernel(xr, srcr, dstr, out, acc, sv, dv, rows, gsa, gsb, ssem,
                  isem):
    c = lax.axis_index("c")
    s = lax.axis_index("s")
    start, nb = _my_batches(c, s, nbw0, nbw1)
    base = s * RPT
    def zb(i, _):
      for v in range(D // 16):
        rows[0, i, pl.ds(v * 16, 16)] = jnp.zeros((16,), jnp.float32)
      return 0
    lax.fori_loop(0, EB, zb, 0)
    for off, ln in _row_chunks():
      pltpu.sync_copy(rows.at[0, pl.ds(0, ln)],
                      acc.at[pl.ds(base + off, ln)])
    plsc.subcore_barrier()
    pltpu.sync_copy(srcr.at[start], sv.at[0])
    pltpu.sync_copy(dstr.at[start], dv.at[0])
    pltpu.sync_copy(srcr.at[start + 1], sv.at[1])
    pltpu.sync_copy(dstr.at[start + 1], dv.at[1])
    pltpu.async_copy(srcr.at[start + 2], sv.at[2], isem)
    pltpu.async_copy(dstr.at[start + 2], dv.at[2], isem)
    pltpu.async_copy(xr.at[sv.at[0]], rows.at[0], gsa)
    pltpu.async_copy(xr.at[sv.at[1]], rows.at[1], gsb)
    def eb(b, _):
      cur = lax.rem(b, 3)
      pre = lax.rem(b + 2, 3)
      i0 = lax.rem(b, 4)
      i2 = lax.rem(b + 2, 4)
      i3 = lax.rem(b + 3, 4)
      even = lax.rem(b, 2) == 0
      @pl.when(even)
      def _():
        pltpu.make_async_copy(xr.at[sv.at[i0]], rows.at[cur], gsa).wait()
      @pl.when(jnp.logical_not(even))
      def _():
        pltpu.make_async_copy(xr.at[sv.at[i0]], rows.at[cur], gsb).wait()
      @pl.when(b > 0)
      def _():
        pltpu.make_async_copy(
            rows.at[pre], acc.at[dv.at[lax.rem(b + 3, 4)]], ssem).wait()
      @pl.when(b + 2 < nb)
      def _():
        pltpu.make_async_copy(srcr.at[start + b + 2], sv.at[i2], isem).wait()
        pltpu.make_async_copy(dstr.at[start + b + 2], dv.at[i2], isem).wait()
        @pl.when(even)
        def _():
          pltpu.async_copy(xr.at[sv.at[i2]], rows.at[pre], gsa)
        @pl.when(jnp.logical_not(even))
        def _():
          pltpu.async_copy(xr.at[sv.at[i2]], rows.at[pre], gsb)
      @pl.when(b + 3 < nb)
      def _():
        pltpu.async_copy(srcr.at[start + b + 3], sv.at[i3], isem)
        pltpu.async_copy(dstr.at[start + b + 3], dv.at[i3], isem)
      pltpu.async_copy(rows.at[cur], acc.at[dv.at[i0]], ssem, add=True)
      return 0
    lax.fori_loop(0, nb, eb, 0)
    pltpu.make_async_copy(
        rows.at[lax.rem(nb - 1, 3)], acc.at[dv.at[lax.rem(nb - 1, 4)]],
        ssem).wait()
    plsc.subcore_barrier()
    for off, ln in _row_chunks():
      pltpu.sync_copy(acc.at[pl.ds(base + off, ln)],
                      rows.at[0, pl.ds(0, ln)])
      pltpu.sync_copy(rows.at[0, pl.ds(0, ln)],
                      out.at[c, pl.ds(base + off, ln)])

  return prop_kernel


def _sc_gat(nbw0, nbw1):
  @functools.partial(
      pl.kernel,
      out_type=(
          jax.ShapeDtypeStruct((NC, NPAD, D), jnp.float32),
          jax.ShapeDtypeStruct((NC, NPAD), jnp.float32),
      ),
      mesh=plsc.VectorSubcoreMesh(**_MESH),
      scratch_types=[
          pltpu.VMEM_SHARED((NPAD, D), jnp.float32),
          pltpu.VMEM_SHARED((NPAD,), jnp.float32),
          pltpu.VMEM((4, EB), jnp.int32),
          pltpu.VMEM((4, EB), jnp.int32),
          pltpu.VMEM((3, EB, D), jnp.float32),
          pltpu.VMEM((3, EB), jnp.float32),
          pltpu.VMEM((3, EB), jnp.float32),
          pltpu.VMEM((EB,), jnp.float32),
          pltpu.VMEM((16,), jnp.float32),
          pltpu.VMEM((RPT,), jnp.float32),
          pltpu.SemaphoreType.DMA,
          pltpu.SemaphoreType.DMA,
          pltpu.SemaphoreType.DMA,
          pltpu.SemaphoreType.DMA,
          pltpu.SemaphoreType.DMA,
          pltpu.SemaphoreType.DMA,
      ],
  )
  def gat_kernel(zr, elr, err, c16r, srcr, dstr, outn, outd,
                 accn, accd, sv, dv, rows, elv, erv, av, cv, stage,
                 gsa, gsb, esa, esb, ssem, isem):
    c = lax.axis_index("c")
    s = lax.axis_index("s")
    start, nb = _my_batches(c, s, nbw0, nbw1)
    base = s * RPT
    def zb(i, _):
      for v in range(D // 16):
        rows[0, i, pl.ds(v * 16, 16)] = jnp.zeros((16,), jnp.float32)
      return 0
    lax.fori_loop(0, EB, zb, 0)
    for off, ln in _row_chunks():
      pltpu.sync_copy(rows.at[0, pl.ds(0, ln)],
                      accn.at[pl.ds(base + off, ln)])
    _zero_vec(stage, RPT)
    pltpu.sync_copy(stage, accd.at[pl.ds(base, RPT)])
    plsc.subcore_barrier()
    pltpu.sync_copy(c16r, cv)
    cvec = cv[...]
    pltpu.sync_copy(srcr.at[start], sv.at[0])
    pltpu.sync_copy(dstr.at[start], dv.at[0])
    pltpu.sync_copy(srcr.at[start + 1], sv.at[1])
    pltpu.sync_copy(dstr.at[start + 1], dv.at[1])
    pltpu.async_copy(srcr.at[start + 2], sv.at[2], isem)
    pltpu.async_copy(dstr.at[start + 2], dv.at[2], isem)
    pltpu.async_copy(elr.at[sv.at[0]], elv.at[0], esa)
    pltpu.async_copy(err.at[dv.at[0]], erv.at[0], esa)
    pltpu.async_copy(zr.at[sv.at[0]], rows.at[0], gsa)
    pltpu.async_copy(elr.at[sv.at[1]], elv.at[1], esb)
    pltpu.async_copy(err.at[dv.at[1]], erv.at[1], esb)
    pltpu.async_copy(zr.at[sv.at[1]], rows.at[1], gsb)
    def eb(b, _):
      cur = lax.rem(b, 3)
      pre = lax.rem(b + 2, 3)
      i0 = lax.rem(b, 4)
      i2 = lax.rem(b + 2, 4)
      i3 = lax.rem(b + 3, 4)
      even = lax.rem(b, 2) == 0
      idxd = dv.at[i0]
      @pl.when(even)
      def _():
        pltpu.make_async_copy(elr.at[sv.at[i0]], elv.at[cur], esa).wait()
        pltpu.make_async_copy(err.at[idxd], erv.at[cur], esa).wait()
      @pl.when(jnp.logical_not(even))
      def _():
        pltpu.make_async_copy(elr.at[sv.at[i0]], elv.at[cur], esb).wait()
        pltpu.make_async_copy(err.at[idxd], erv.at[cur], esb).wait()
      for k in range(EB // 16):
        sm = elv[cur, pl.ds(k * 16, 16)] + erv[cur, pl.ds(k * 16, 16)]
        lr = jnp.where(sm > 0, sm, 0.2 * sm)
        av[pl.ds(k * 16, 16)] = jnp.exp(lr - cvec)
      pltpu.sync_copy(av, accd.at[idxd], add=True)
      @pl.when(even)
      def _():
        pltpu.make_async_copy(zr.at[sv.at[i0]], rows.at[cur], gsa).wait()
      @pl.when(jnp.logical_not(even))
      def _():
        pltpu.make_async_copy(zr.at[sv.at[i0]], rows.at[cur], gsb).wait()
      @pl.when(b > 0)
      def _():
        pltpu.make_async_copy(
            rows.at[pre], accn.at[dv.at[lax.rem(b + 3, 4)]], ssem).wait()
      @pl.when(b + 2 < nb)
      def _():
        pltpu.make_async_copy(srcr.at[start + b + 2], sv.at[i2], isem).wait()
        pltpu.make_async_copy(dstr.at[start + b + 2], dv.at[i2], isem).wait()
        @pl.when(even)
        def _():
          pltpu.async_copy(elr.at[sv.at[i2]], elv.at[pre], esa)
          pltpu.async_copy(err.at[dv.at[i2]], erv.at[pre], esa)
          pltpu.async_copy(zr.at[sv.at[i2]], rows.at[pre], gsa)
        @pl.when(jnp.logical_not(even))
        def _():
          pltpu.async_copy(elr.at[sv.at[i2]], elv.at[pre], esb)
          pltpu.async_copy(err.at[dv.at[i2]], erv.at[pre], esb)
          pltpu.async_copy(zr.at[sv.at[i2]], rows.at[pre], gsb)
      @pl.when(b + 3 < nb)
      def _():
        pltpu.async_copy(srcr.at[start + b + 3], sv.at[i3], isem)
        pltpu.async_copy(dstr.at[start + b + 3], dv.at[i3], isem)
      for g in range(EB // 16):
        a16 = av[pl.ds(g * 16, 16)]
        def sb(l, _):
          asp = jnp.take_along_axis(
              a16, lax.broadcast(l, (16,)), axis=0,
              mode="promise_in_bounds")
          row = g * 16 + l
          for v in range(D // 16):
            rows[cur, row, pl.ds(v * 16, 16)] = (
                rows[cur, row, pl.ds(v * 16, 16)] * asp)
          return 0
        lax.fori_loop(0, 16, sb, 0)
      pltpu.async_copy(rows.at[cur], accn.at[idxd], ssem, add=True)
      return 0
    lax.fori_loop(0, nb, eb, 0)
    pltpu.make_async_copy(
        rows.at[lax.rem(nb - 1, 3)], accn.at[dv.at[lax.rem(nb - 1, 4)]],
        ssem).wait()
    plsc.subcore_barrier()
    for off, ln in _row_chunks():
      pltpu.sync_copy(accn.at[pl.ds(base + off, ln)],
                      rows.at[0, pl.ds(0, ln)])
      pltpu.sync_copy(rows.at[0, pl.ds(0, ln)],
                      outn.at[c, pl.ds(base + off, ln)])
    pltpu.sync_copy(accd.at[pl.ds(base, RPT)], stage)
    pltpu.sync_copy(stage, outd.at[c, pl.ds(base, RPT)])

  return gat_kernel


_BLKG = 1024


def _tc_gat_fn(y_ref, w_ref, al_ref, ar_ref, z_ref, el_ref, er_ref, cm_ref):
  i = pl.program_id(0)
  z = jnp.dot(y_ref[...], w_ref[...], preferred_element_type=jnp.float32)
  z_ref[...] = z
  el = jnp.sum(z * al_ref[...], axis=1, keepdims=True)
  er = jnp.sum(z * ar_ref[...], axis=1, keepdims=True)
  el_ref[...] = el
  er_ref[...] = er
  vals = jnp.concatenate(
      [jnp.full((1, 128), jnp.max(el), jnp.float32),
       jnp.full((1, 128), jnp.max(er), jnp.float32)], axis=0)
  @pl.when(i == 0)
  def _():
    cm_ref[...] = vals
  @pl.when(i > 0)
  def _():
    cm_ref[...] = jnp.maximum(cm_ref[...], vals)


def _tc_gat(y, W, al, ar):
  return pl.pallas_call(
      _tc_gat_fn,
      grid=(NPAD // _BLKG,),
      in_specs=[
          pl.BlockSpec((_BLKG, D), lambda i: (i, 0)),
          pl.BlockSpec((D, D), lambda i: (0, 0)),
          pl.BlockSpec((1, D), lambda i: (0, 0)),
          pl.BlockSpec((1, D), lambda i: (0, 0)),
      ],
      out_specs=[
          pl.BlockSpec((_BLKG, D), lambda i: (i, 0)),
          pl.BlockSpec((_BLKG, 1), lambda i: (i, 0)),
          pl.BlockSpec((_BLKG, 1), lambda i: (i, 0)),
          pl.BlockSpec((2, 128), lambda i: (0, 0)),
      ],
      out_shape=[
          jax.ShapeDtypeStruct((NPAD, D), jnp.float32),
          jax.ShapeDtypeStruct((NPAD, 1), jnp.float32),
          jax.ShapeDtypeStruct((NPAD, 1), jnp.float32),
          jax.ShapeDtypeStruct((2, 128), jnp.float32),
      ],
  )(y, W, al.reshape(1, D), ar.reshape(1, D))


def _tc_final_fn(num_ref, den_ref, b_ref, o_ref):
  den = den_ref[...]
  rst = num_ref[...] / jnp.where(den > 0, den, 1.0)
  t = jnp.maximum(rst + b_ref[...], 0.0)
  m = jnp.max(t, axis=1, keepdims=True)
  lse = jnp.log(jnp.sum(jnp.exp(t - m), axis=1, keepdims=True))
  o_ref[...] = t - m - lse


def _tc_final(num, den, bias):
  return pl.pallas_call(
      _tc_final_fn,
      grid=(NPAD // _BLKG,),
      in_specs=[
          pl.BlockSpec((_BLKG, D), lambda i: (i, 0)),
          pl.BlockSpec((_BLKG, 1), lambda i: (i, 0)),
          pl.BlockSpec((1, D), lambda i: (0, 0)),
      ],
      out_specs=pl.BlockSpec((_BLKG, D), lambda i: (i, 0)),
      out_shape=jax.ShapeDtypeStruct((NPAD, D), jnp.float32),
  )(num, den, bias.reshape(1, D))


def kernel(feats, edge_index, W, attn_l, attn_r, bias):
  n, d = feats.shape
  e = edge_index.shape[1]
  ep = -(-e // (NW * EB)) * (NW * EB)
  nbt = ep // EB
  nbw0 = round(nbt * 0.82 / NS)
  nbw1 = nbt // NS - nbw0
  nbw0g = round(nbt * 0.75 / NS)
  nbw1g = nbt // NS - nbw0g

  src = edge_index[0]
  dst = edge_index[1]
  pad = ep - e
  padv = jnp.full((pad,), n, jnp.int32)
  srcp = jnp.concatenate([src, padv]).reshape(nbt, EB)
  dstp = jnp.concatenate([dst, padv]).reshape(nbt, EB)
  feats_p = jnp.zeros((NPAD, d), feats.dtype).at[:n].set(feats)

  degs = _sc_degree(nbt // NS // 2, nbt // NS - nbt // NS // 2)(srcp, dstp)
  out_deg = degs[0, 0] + degs[1, 0]
  in_deg = degs[0, 1] + degs[1, 1]
  norm1 = jax.lax.rsqrt(jnp.clip(in_deg, 1.0))
  norm2 = jax.lax.rsqrt(jnp.clip(out_deg, 1.0))
  s11 = (norm1 * norm1)[:, None]
  s12 = (norm1 * norm2)[:, None]
  n2c = norm2[:, None]

  prop = _sc_prop(nbw0, nbw1)
  raw1 = prop(feats_p * n2c, srcp, dstp)
  in2 = (raw1[0] + raw1[1]) * s11
  raw2 = prop(in2, srcp, dstp)
  r2 = raw2[0] + raw2[1]
  y = feats_p + r2 * n2c
  raw3 = prop(r2 * s12, srcp, dstp)
  r3 = raw3[0] + raw3[1]
  y = y + r3 * n2c
  raw4 = prop(r3 * s12, srcp, dstp)
  r4 = raw4[0] + raw4[1]
  y = (y + r4 * n2c) * 0.25

  z, el, er, cm = _tc_gat(y, W, attn_l, attn_r)
  c16 = jnp.full((16,), cm[0, 0] + cm[1, 0], jnp.float32)
  nump, denp = _sc_gat(nbw0g, nbw1g)(
      z, el.reshape(NPAD), er.reshape(NPAD), c16, srcp, dstp)
  num = nump[0] + nump[1]
  den = (denp[0] + denp[1])[:, None]
  logp = _tc_final(num, den, bias)
  return logp[:n]


# speedup vs baseline: 1.4489x; 1.4489x over previous
import functools

import jax
import jax.numpy as jnp
from jax import lax
from jax.experimental import pallas as pl
from jax.experimental.pallas import tpu as pltpu
from jax.experimental.pallas import tpu_sc as plsc

N = 10000
D = 128
NC = 2
NS = 16
NW = NC * NS
EB = 112
NPAD = 10240
RPT = NPAD // NS

_MESH = dict(core_axis_name="c", subcore_axis_name="s")


def _my_batches(c, s, nbw0, nbw1):
  start = jnp.where(c == 0, s * nbw0, NS * nbw0 + s * nbw1)
  nb = jnp.where(c == 0, nbw0, nbw1)
  return start, nb


def _row_chunks():
  chunks = []
  off = 0
  while off < RPT:
    ln = min(EB, RPT - off)
    chunks.append((off, ln))
    off += ln
  return chunks


def _zero_rows(rows):
  def zb(i, _):
    for v in range(D // 16):
      rows[i, pl.ds(v * 16, 16)] = jnp.zeros((16,), jnp.float32)
    return 0
  lax.fori_loop(0, EB, zb, 0)


def _zero_vec(vec, n):
  def zb(i, _):
    vec[pl.ds(i * 16, 16)] = jnp.zeros((16,), jnp.float32)
    return 0
  lax.fori_loop(0, n // 16, zb, 0)


def _sc_degree(nbw0, nbw1):
  @functools.partial(
      pl.kernel,
      out_type=jax.ShapeDtypeStruct((NC, 2, NPAD), jnp.float32),
      mesh=plsc.VectorSubcoreMesh(**_MESH),
      scratch_types=[
          pltpu.VMEM_SHARED((NPAD,), jnp.float32),
          pltpu.VMEM_SHARED((NPAD,), jnp.float32),
          pltpu.VMEM((4, EB), jnp.int32),
          pltpu.VMEM((4, EB), jnp.int32),
          pltpu.VMEM((EB,), jnp.float32),
          pltpu.VMEM((RPT,), jnp.float32),
          pltpu.SemaphoreType.DMA,
      ],
  )
  def deg_kernel(srcr, dstr, out, acc_s, acc_d, sv, dv, ones, stage, isem):
    c = lax.axis_index("c")
    s = lax.axis_index("s")
    start, nb = _my_batches(c, s, nbw0, nbw1)
    base = s * RPT
    _zero_vec(stage, RPT)
    def ob(i, _):
      ones[pl.ds(i * 16, 16)] = jnp.ones((16,), jnp.float32)
      return 0
    lax.fori_loop(0, EB // 16, ob, 0)
    pltpu.sync_copy(stage, acc_s.at[pl.ds(base, RPT)])
    pltpu.sync_copy(stage, acc_d.at[pl.ds(base, RPT)])
    plsc.subcore_barrier()
    pltpu.sync_copy(srcr.at[start], sv.at[0])
    pltpu.sync_copy(dstr.at[start], dv.at[0])
    pltpu.async_copy(srcr.at[start + 1], sv.at[1], isem)
    pltpu.async_copy(dstr.at[start + 1], dv.at[1], isem)
    def eb(b, _):
      i0 = lax.rem(b, 4)
      i1 = lax.rem(b + 1, 4)
      pltpu.sync_copy(ones, acc_s.at[sv.at[i0]], add=True)
      pltpu.sync_copy(ones, acc_d.at[dv.at[i0]], add=True)
      @pl.when(b + 1 < nb)
      def _():
        pltpu.make_async_copy(srcr.at[start + b + 1], sv.at[i1], isem).wait()
        pltpu.make_async_copy(dstr.at[start + b + 1], dv.at[i1], isem).wait()
      @pl.when(b + 2 < nb)
      def _():
        i2 = lax.rem(b + 2, 4)
        pltpu.async_copy(srcr.at[start + b + 2], sv.at[i2], isem)
        pltpu.async_copy(dstr.at[start + b + 2], dv.at[i2], isem)
      return 0
    lax.fori_loop(0, nb, eb, 0)
    plsc.subcore_barrier()
    pltpu.sync_copy(acc_s.at[pl.ds(base, RPT)], stage)
    pltpu.sync_copy(stage, out.at[c, 0, pl.ds(base, RPT)])
    pltpu.sync_copy(acc_d.at[pl.ds(base, RPT)], stage)
    pltpu.sync_copy(stage, out.at[c, 1, pl.ds(base, RPT)])

  return deg_kernel


def _sc_prop(nbw0, nbw1):
  @functools.partial(
      pl.kernel,
      out_type=jax.ShapeDtypeStruct((NC, NPAD, D), jnp.float32),
      mesh=plsc.VectorSubcoreMesh(**_MESH),
      scratch_types=[
          pltpu.VMEM_SHARED((NPAD, D), jnp.float32),
          pltpu.VMEM((4, EB), jnp.int32),
          pltpu.VMEM((4, EB), jnp.int32),
          pltpu.VMEM((3, EB, D), jnp.float32),
          pltpu.SemaphoreType.DMA,
          pltpu.SemaphoreType.DMA,
          pltpu.SemaphoreType.DMA,
          pltpu.SemaphoreType.DMA,
      ],
  )
  def prop_kernel(xr, srcr, dstr, out, acc, sv, dv, rows, gsa, gsb, ssem,
                  isem):
    c = lax.axis_index("c")
    s = lax.axis_index("s")
    start, nb = _my_batches(c, s, nbw0, nbw1)
    base = s * RPT
    def zb(i, _):
      for v in range(D // 16):
        rows[0, i, pl.ds(v * 16, 16)] = jnp.zeros((16,), jnp.float32)
      return 0
    lax.fori_loop(0, EB, zb, 0)
    for off, ln in _row_chunks():
      pltpu.sync_copy(rows.at[0, pl.ds(0, ln)],
                      acc.at[pl.ds(base + off, ln)])
    plsc.subcore_barrier()
    pltpu.sync_copy(srcr.at[start], sv.at[0])
    pltpu.sync_copy(dstr.at[start], dv.at[0])
    pltpu.sync_copy(srcr.at[start + 1], sv.at[1])
    pltpu.sync_copy(dstr.at[start + 1], dv.at[1])
    pltpu.async_copy(srcr.at[start + 2], sv.at[2], isem)
    pltpu.async_copy(dstr.at[start + 2], dv.at[2], isem)
    pltpu.async_copy(xr.at[sv.at[0]], rows.at[0], gsa)
    pltpu.async_copy(xr.at[sv.at[1]], rows.at[1], gsb)
    def eb(b, _):
      cur = lax.rem(b, 3)
      pre = lax.rem(b + 2, 3)
      i0 = lax.rem(b, 4)
      i2 = lax.rem(b + 2, 4)
      i3 = lax.rem(b + 3, 4)
      even = lax.rem(b, 2) == 0
      @pl.when(even)
      def _():
        pltpu.make_async_copy(xr.at[sv.at[i0]], rows.at[cur], gsa).wait()
      @pl.when(jnp.logical_not(even))
      def _():
        pltpu.make_async_copy(xr.at[sv.at[i0]], rows.at[cur], gsb).wait()
      @pl.when(b > 0)
      def _():
        pltpu.make_async_copy(
            rows.at[pre], acc.at[dv.at[lax.rem(b + 3, 4)]], ssem).wait()
      @pl.when(b + 2 < nb)
      def _():
        pltpu.make_async_copy(srcr.at[start + b + 2], sv.at[i2], isem).wait()
        pltpu.make_async_copy(dstr.at[start + b + 2], dv.at[i2], isem).wait()
        @pl.when(even)
        def _():
          pltpu.async_copy(xr.at[sv.at[i2]], rows.at[pre], gsa)
        @pl.when(jnp.logical_not(even))
        def _():
          pltpu.async_copy(xr.at[sv.at[i2]], rows.at[pre], gsb)
      @pl.when(b + 3 < nb)
      def _():
        pltpu.async_copy(srcr.at[start + b + 3], sv.at[i3], isem)
        pltpu.async_copy(dstr.at[start + b + 3], dv.at[i3], isem)
      pltpu.async_copy(rows.at[cur], acc.at[dv.at[i0]], ssem, add=True)
      return 0
    lax.fori_loop(0, nb, eb, 0)
    pltpu.make_async_copy(
        rows.at[lax.rem(nb - 1, 3)], acc.at[dv.at[lax.rem(nb - 1, 4)]],
        ssem).wait()
    plsc.subcore_barrier()
    for off, ln in _row_chunks():
      pltpu.sync_copy(acc.at[pl.ds(base + off, ln)],
                      rows.at[0, pl.ds(0, ln)])
      pltpu.sync_copy(rows.at[0, pl.ds(0, ln)],
                      out.at[c, pl.ds(base + off, ln)])

  return prop_kernel


def _sc_gat(nbw0, nbw1):
  @functools.partial(
      pl.kernel,
      out_type=(
          jax.ShapeDtypeStruct((NC, NPAD, D), jnp.float32),
          jax.ShapeDtypeStruct((NC, NPAD), jnp.float32),
      ),
      mesh=plsc.VectorSubcoreMesh(**_MESH),
      scratch_types=[
          pltpu.VMEM_SHARED((NPAD, D), jnp.float32),
          pltpu.VMEM_SHARED((NPAD,), jnp.float32),
          pltpu.VMEM((4, EB), jnp.int32),
          pltpu.VMEM((4, EB), jnp.int32),
          pltpu.VMEM((2, EB, D), jnp.float32),
          pltpu.VMEM((2, EB), jnp.float32),
          pltpu.VMEM((2, EB), jnp.float32),
          pltpu.VMEM((EB,), jnp.float32),
          pltpu.VMEM((16,), jnp.float32),
          pltpu.VMEM((RPT,), jnp.float32),
          pltpu.SemaphoreType.DMA,
          pltpu.SemaphoreType.DMA,
          pltpu.SemaphoreType.DMA,
          pltpu.SemaphoreType.DMA,
      ],
  )
  def gat_kernel(zr, elr, err, c16r, srcr, dstr, outn, outd,
                 accn, accd, sv, dv, rows, elv, erv, av, cv, stage,
                 gsem, ssem, esem, isem):
    c = lax.axis_index("c")
    s = lax.axis_index("s")
    start, nb = _my_batches(c, s, nbw0, nbw1)
    base = s * RPT
    def zb(i, _):
      for v in range(D // 16):
        rows[0, i, pl.ds(v * 16, 16)] = jnp.zeros((16,), jnp.float32)
      return 0
    lax.fori_loop(0, EB, zb, 0)
    for off, ln in _row_chunks():
      pltpu.sync_copy(rows.at[0, pl.ds(0, ln)],
                      accn.at[pl.ds(base + off, ln)])
    _zero_vec(stage, RPT)
    pltpu.sync_copy(stage, accd.at[pl.ds(base, RPT)])
    plsc.subcore_barrier()
    pltpu.sync_copy(c16r, cv)
    cvec = cv[...]
    pltpu.sync_copy(srcr.at[start], sv.at[0])
    pltpu.sync_copy(dstr.at[start], dv.at[0])
    pltpu.async_copy(srcr.at[start + 1], sv.at[1], isem)
    pltpu.async_copy(dstr.at[start + 1], dv.at[1], isem)
    pltpu.async_copy(elr.at[sv.at[0]], elv.at[0], esem)
    pltpu.async_copy(err.at[dv.at[0]], erv.at[0], esem)
    pltpu.async_copy(zr.at[sv.at[0]], rows.at[0], gsem)
    def eb(b, _):
      cur = lax.rem(b, 2)
      nxt = 1 - cur
      i0 = lax.rem(b, 4)
      i1 = lax.rem(b + 1, 4)
      i2 = lax.rem(b + 2, 4)
      idxd = dv.at[i0]
      pltpu.make_async_copy(elr.at[sv.at[i0]], elv.at[cur], esem).wait()
      pltpu.make_async_copy(err.at[idxd], erv.at[cur], esem).wait()
      for k in range(EB // 16):
        sm = elv[cur, pl.ds(k * 16, 16)] + erv[cur, pl.ds(k * 16, 16)]
        lr = jnp.where(sm > 0, sm, 0.2 * sm)
        av[pl.ds(k * 16, 16)] = jnp.exp(lr - cvec)
      pltpu.sync_copy(av, accd.at[idxd], add=True)
      pltpu.make_async_copy(zr.at[sv.at[i0]], rows.at[cur], gsem).wait()
      @pl.when(b > 0)
      def _():
        pltpu.make_async_copy(
            rows.at[nxt], accn.at[dv.at[lax.rem(b + 3, 4)]], ssem).wait()
      @pl.when(b + 1 < nb)
      def _():
        pltpu.make_async_copy(srcr.at[start + b + 1], sv.at[i1], isem).wait()
        pltpu.make_async_copy(dstr.at[start + b + 1], dv.at[i1], isem).wait()
        pltpu.async_copy(elr.at[sv.at[i1]], elv.at[nxt], esem)
        pltpu.async_copy(err.at[dv.at[i1]], erv.at[nxt], esem)
        pltpu.async_copy(zr.at[sv.at[i1]], rows.at[nxt], gsem)
      @pl.when(b + 2 < nb)
      def _():
        pltpu.async_copy(srcr.at[start + b + 2], sv.at[i2], isem)
        pltpu.async_copy(dstr.at[start + b + 2], dv.at[i2], isem)
      for g in range(EB // 16):
        a16 = av[pl.ds(g * 16, 16)]
        def sb(l, _):
          asp = jnp.take_along_axis(
              a16, lax.broadcast(l, (16,)), axis=0,
              mode="promise_in_bounds")
          row = g * 16 + l
          for v in range(D // 16):
            rows[cur, row, pl.ds(v * 16, 16)] = (
                rows[cur, row, pl.ds(v * 16, 16)] * asp)
          return 0
        lax.fori_loop(0, 16, sb, 0)
      pltpu.async_copy(rows.at[cur], accn.at[idxd], ssem, add=True)
      return 0
    lax.fori_loop(0, nb, eb, 0)
    pltpu.make_async_copy(
        rows.at[lax.rem(nb - 1, 2)], accn.at[dv.at[lax.rem(nb - 1, 4)]],
        ssem).wait()
    plsc.subcore_barrier()
    for off, ln in _row_chunks():
      pltpu.sync_copy(accn.at[pl.ds(base + off, ln)],
                      rows.at[0, pl.ds(0, ln)])
      pltpu.sync_copy(rows.at[0, pl.ds(0, ln)],
                      outn.at[c, pl.ds(base + off, ln)])
    pltpu.sync_copy(accd.at[pl.ds(base, RPT)], stage)
    pltpu.sync_copy(stage, outd.at[c, pl.ds(base, RPT)])

  return gat_kernel


_BLKG = 1024


def _tc_gat_fn(y_ref, w_ref, al_ref, ar_ref, z_ref, el_ref, er_ref, cm_ref):
  i = pl.program_id(0)
  z = jnp.dot(y_ref[...], w_ref[...], preferred_element_type=jnp.float32)
  z_ref[...] = z
  el = jnp.sum(z * al_ref[...], axis=1, keepdims=True)
  er = jnp.sum(z * ar_ref[...], axis=1, keepdims=True)
  el_ref[...] = el
  er_ref[...] = er
  vals = jnp.concatenate(
      [jnp.full((1, 128), jnp.max(el), jnp.float32),
       jnp.full((1, 128), jnp.max(er), jnp.float32)], axis=0)
  @pl.when(i == 0)
  def _():
    cm_ref[...] = vals
  @pl.when(i > 0)
  def _():
    cm_ref[...] = jnp.maximum(cm_ref[...], vals)


def _tc_gat(y, W, al, ar):
  return pl.pallas_call(
      _tc_gat_fn,
      grid=(NPAD // _BLKG,),
      in_specs=[
          pl.BlockSpec((_BLKG, D), lambda i: (i, 0)),
          pl.BlockSpec((D, D), lambda i: (0, 0)),
          pl.BlockSpec((1, D), lambda i: (0, 0)),
          pl.BlockSpec((1, D), lambda i: (0, 0)),
      ],
      out_specs=[
          pl.BlockSpec((_BLKG, D), lambda i: (i, 0)),
          pl.BlockSpec((_BLKG, 1), lambda i: (i, 0)),
          pl.BlockSpec((_BLKG, 1), lambda i: (i, 0)),
          pl.BlockSpec((2, 128), lambda i: (0, 0)),
      ],
      out_shape=[
          jax.ShapeDtypeStruct((NPAD, D), jnp.float32),
          jax.ShapeDtypeStruct((NPAD, 1), jnp.float32),
          jax.ShapeDtypeStruct((NPAD, 1), jnp.float32),
          jax.ShapeDtypeStruct((2, 128), jnp.float32),
      ],
  )(y, W, al.reshape(1, D), ar.reshape(1, D))


def _tc_final_fn(num_ref, den_ref, b_ref, o_ref):
  den = den_ref[...]
  rst = num_ref[...] / jnp.where(den > 0, den, 1.0)
  t = jnp.maximum(rst + b_ref[...], 0.0)
  m = jnp.max(t, axis=1, keepdims=True)
  lse = jnp.log(jnp.sum(jnp.exp(t - m), axis=1, keepdims=True))
  o_ref[...] = t - m - lse


def _tc_final(num, den, bias):
  return pl.pallas_call(
      _tc_final_fn,
      grid=(NPAD // _BLKG,),
      in_specs=[
          pl.BlockSpec((_BLKG, D), lambda i: (i, 0)),
          pl.BlockSpec((_BLKG, 1), lambda i: (i, 0)),
          pl.BlockSpec((1, D), lambda i: (0, 0)),
      ],
      out_specs=pl.BlockSpec((_BLKG, D), lambda i: (i, 0)),
      out_shape=jax.ShapeDtypeStruct((NPAD, D), jnp.float32),
  )(num, den, bias.reshape(1, D))


def kernel(feats, edge_index, W, attn_l, attn_r, bias):
  n, d = feats.shape
  e = edge_index.shape[1]
  ep = -(-e // (NW * EB)) * (NW * EB)
  nbt = ep // EB
  nbw0 = round(nbt * 0.82 / NS)
  nbw1 = nbt // NS - nbw0
  nbw0g = round(nbt * 0.75 / NS)
  nbw1g = nbt // NS - nbw0g

  src = edge_index[0]
  dst = edge_index[1]
  pad = ep - e
  padv = jnp.full((pad,), n, jnp.int32)
  srcp = jnp.concatenate([src, padv]).reshape(nbt, EB)
  dstp = jnp.concatenate([dst, padv]).reshape(nbt, EB)
  feats_p = jnp.zeros((NPAD, d), feats.dtype).at[:n].set(feats)

  degs = _sc_degree(nbt // NS // 2, nbt // NS - nbt // NS // 2)(srcp, dstp)
  out_deg = degs[0, 0] + degs[1, 0]
  in_deg = degs[0, 1] + degs[1, 1]
  norm1 = jax.lax.rsqrt(jnp.clip(in_deg, 1.0))
  norm2 = jax.lax.rsqrt(jnp.clip(out_deg, 1.0))
  s11 = (norm1 * norm1)[:, None]
  s12 = (norm1 * norm2)[:, None]
  n2c = norm2[:, None]

  prop = _sc_prop(nbw0, nbw1)
  raw1 = prop(feats_p * n2c, srcp, dstp)
  in2 = (raw1[0] + raw1[1]) * s11
  raw2 = prop(in2, srcp, dstp)
  r2 = raw2[0] + raw2[1]
  y = feats_p + r2 * n2c
  raw3 = prop(r2 * s12, srcp, dstp)
  r3 = raw3[0] + raw3[1]
  y = y + r3 * n2c
  raw4 = prop(r3 * s12, srcp, dstp)
  r4 = raw4[0] + raw4[1]
  y = (y + r4 * n2c) * 0.25

  z, el, er, cm = _tc_gat(y, W, attn_l, attn_r)
  c16 = jnp.full((16,), cm[0, 0] + cm[1, 0], jnp.float32)
  nump, denp = _sc_gat(nbw0g, nbw1g)(
      z, el.reshape(NPAD), er.reshape(NPAD), c16, srcp, dstp)
  num = nump[0] + nump[1]
  den = (denp[0] + denp[1])[:, None]
  logp = _tc_final(num, den, bias)
  return logp[:n]


# speedup vs baseline: 1.4631x; 1.0099x over previous
import functools

import jax
import jax.numpy as jnp
from jax import lax
from jax.experimental import pallas as pl
from jax.experimental.pallas import tpu as pltpu
from jax.experimental.pallas import tpu_sc as plsc

N = 10000
D = 128
NC = 2
NS = 16
NW = NC * NS
EB = 112
NPAD = 10240
RPT = NPAD // NS

_MESH = dict(core_axis_name="c", subcore_axis_name="s")


def _my_batches(c, s, nbw0, nbw1):
  start = jnp.where(c == 0, s * nbw0, NS * nbw0 + s * nbw1)
  nb = jnp.where(c == 0, nbw0, nbw1)
  return start, nb


def _row_chunks():
  chunks = []
  off = 0
  while off < RPT:
    ln = min(EB, RPT - off)
    chunks.append((off, ln))
    off += ln
  return chunks


def _zero_rows(rows):
  def zb(i, _):
    for v in range(D // 16):
      rows[i, pl.ds(v * 16, 16)] = jnp.zeros((16,), jnp.float32)
    return 0
  lax.fori_loop(0, EB, zb, 0)


def _zero_vec(vec, n):
  def zb(i, _):
    vec[pl.ds(i * 16, 16)] = jnp.zeros((16,), jnp.float32)
    return 0
  lax.fori_loop(0, n // 16, zb, 0)


def _sc_degree(nbw0, nbw1):
  @functools.partial(
      pl.kernel,
      out_type=jax.ShapeDtypeStruct((NC, 2, NPAD), jnp.float32),
      mesh=plsc.VectorSubcoreMesh(**_MESH),
      scratch_types=[
          pltpu.VMEM_SHARED((NPAD,), jnp.float32),
          pltpu.VMEM_SHARED((NPAD,), jnp.float32),
          pltpu.VMEM((4, EB), jnp.int32),
          pltpu.VMEM((4, EB), jnp.int32),
          pltpu.VMEM((EB,), jnp.float32),
          pltpu.VMEM((RPT,), jnp.float32),
          pltpu.SemaphoreType.DMA,
      ],
  )
  def deg_kernel(srcr, dstr, out, acc_s, acc_d, sv, dv, ones, stage, isem):
    c = lax.axis_index("c")
    s = lax.axis_index("s")
    start, nb = _my_batches(c, s, nbw0, nbw1)
    base = s * RPT
    _zero_vec(stage, RPT)
    def ob(i, _):
      ones[pl.ds(i * 16, 16)] = jnp.ones((16,), jnp.float32)
      return 0
    lax.fori_loop(0, EB // 16, ob, 0)
    pltpu.sync_copy(stage, acc_s.at[pl.ds(base, RPT)])
    pltpu.sync_copy(stage, acc_d.at[pl.ds(base, RPT)])
    plsc.subcore_barrier()
    pltpu.sync_copy(srcr.at[start], sv.at[0])
    pltpu.sync_copy(dstr.at[start], dv.at[0])
    pltpu.async_copy(srcr.at[start + 1], sv.at[1], isem)
    pltpu.async_copy(dstr.at[start + 1], dv.at[1], isem)
    def eb(b, _):
      i0 = lax.rem(b, 4)
      i1 = lax.rem(b + 1, 4)
      pltpu.sync_copy(ones, acc_s.at[sv.at[i0]], add=True)
      pltpu.sync_copy(ones, acc_d.at[dv.at[i0]], add=True)
      @pl.when(b + 1 < nb)
      def _():
        pltpu.make_async_copy(srcr.at[start + b + 1], sv.at[i1], isem).wait()
        pltpu.make_async_copy(dstr.at[start + b + 1], dv.at[i1], isem).wait()
      @pl.when(b + 2 < nb)
      def _():
        i2 = lax.rem(b + 2, 4)
        pltpu.async_copy(srcr.at[start + b + 2], sv.at[i2], isem)
        pltpu.async_copy(dstr.at[start + b + 2], dv.at[i2], isem)
      return 0
    lax.fori_loop(0, nb, eb, 0)
    plsc.subcore_barrier()
    pltpu.sync_copy(acc_s.at[pl.ds(base, RPT)], stage)
    pltpu.sync_copy(stage, out.at[c, 0, pl.ds(base, RPT)])
    pltpu.sync_copy(acc_d.at[pl.ds(base, RPT)], stage)
    pltpu.sync_copy(stage, out.at[c, 1, pl.ds(base, RPT)])

  return deg_kernel


def _sc_prop(nbw0, nbw1):
  @functools.partial(
      pl.kernel,
      out_type=jax.ShapeDtypeStruct((NC, NPAD, D), jnp.float32),
      mesh=plsc.VectorSubcoreMesh(**_MESH),
      scratch_types=[
          pltpu.VMEM_SHARED((NPAD, D), jnp.float32),
          pltpu.VMEM((4, EB), jnp.int32),
          pltpu.VMEM((4, EB), jnp.int32),
          pltpu.VMEM((3, EB, D), jnp.float32),
          pltpu.SemaphoreType.DMA,
          pltpu.SemaphoreType.DMA,
          pltpu.SemaphoreType.DMA,
          pltpu.SemaphoreType.DMA,
      ],
  )
  def prop_kernel(xr, srcr, dstr, out, acc, sv, dv, rows, gsa, gsb, ssem,
                  isem):
    c = lax.axis_index("c")
    s = lax.axis_index("s")
    start, nb = _my_batches(c, s, nbw0, nbw1)
    base = s * RPT
    def zb(i, _):
      for v in range(D // 16):
        rows[0, i, pl.ds(v * 16, 16)] = jnp.zeros((16,), jnp.float32)
      return 0
    lax.fori_loop(0, EB, zb, 0)
    for off, ln in _row_chunks():
      pltpu.sync_copy(rows.at[0, pl.ds(0, ln)],
                      acc.at[pl.ds(base + off, ln)])
    plsc.subcore_barrier()
    pltpu.sync_copy(srcr.at[start], sv.at[0])
    pltpu.sync_copy(dstr.at[start], dv.at[0])
    pltpu.sync_copy(srcr.at[start + 1], sv.at[1])
    pltpu.sync_copy(dstr.at[start + 1], dv.at[1])
    pltpu.async_copy(srcr.at[start + 2], sv.at[2], isem)
    pltpu.async_copy(dstr.at[start + 2], dv.at[2], isem)
    pltpu.async_copy(xr.at[sv.at[0]], rows.at[0], gsa)
    pltpu.async_copy(xr.at[sv.at[1]], rows.at[1], gsb)
    def eb(b, _):
      cur = lax.rem(b, 3)
      pre = lax.rem(b + 2, 3)
      i0 = lax.rem(b, 4)
      i2 = lax.rem(b + 2, 4)
      i3 = lax.rem(b + 3, 4)
      even = lax.rem(b, 2) == 0
      @pl.when(even)
      def _():
        pltpu.make_async_copy(xr.at[sv.at[i0]], rows.at[cur], gsa).wait()
      @pl.when(jnp.logical_not(even))
      def _():
        pltpu.make_async_copy(xr.at[sv.at[i0]], rows.at[cur], gsb).wait()
      @pl.when(b > 0)
      def _():
        pltpu.make_async_copy(
            rows.at[pre], acc.at[dv.at[lax.rem(b + 3, 4)]], ssem).wait()
      @pl.when(b + 2 < nb)
      def _():
        pltpu.make_async_copy(srcr.at[start + b + 2], sv.at[i2], isem).wait()
        pltpu.make_async_copy(dstr.at[start + b + 2], dv.at[i2], isem).wait()
        @pl.when(even)
        def _():
          pltpu.async_copy(xr.at[sv.at[i2]], rows.at[pre], gsa)
        @pl.when(jnp.logical_not(even))
        def _():
          pltpu.async_copy(xr.at[sv.at[i2]], rows.at[pre], gsb)
      @pl.when(b + 3 < nb)
      def _():
        pltpu.async_copy(srcr.at[start + b + 3], sv.at[i3], isem)
        pltpu.async_copy(dstr.at[start + b + 3], dv.at[i3], isem)
      pltpu.async_copy(rows.at[cur], acc.at[dv.at[i0]], ssem, add=True)
      return 0
    lax.fori_loop(0, nb, eb, 0)
    pltpu.make_async_copy(
        rows.at[lax.rem(nb - 1, 3)], acc.at[dv.at[lax.rem(nb - 1, 4)]],
        ssem).wait()
    plsc.subcore_barrier()
    for off, ln in _row_chunks():
      pltpu.sync_copy(acc.at[pl.ds(base + off, ln)],
                      rows.at[0, pl.ds(0, ln)])
      pltpu.sync_copy(rows.at[0, pl.ds(0, ln)],
                      out.at[c, pl.ds(base + off, ln)])

  return prop_kernel


def _sc_gat(nbw0, nbw1):
  @functools.partial(
      pl.kernel,
      out_type=(
          jax.ShapeDtypeStruct((NC, NPAD, D), jnp.float32),
          jax.ShapeDtypeStruct((NC, NPAD), jnp.float32),
      ),
      mesh=plsc.VectorSubcoreMesh(**_MESH),
      scratch_types=[
          pltpu.VMEM_SHARED((NPAD, D), jnp.float32),
          pltpu.VMEM_SHARED((NPAD,), jnp.float32),
          pltpu.VMEM((4, EB), jnp.int32),
          pltpu.VMEM((4, EB), jnp.int32),
          pltpu.VMEM((2, EB, D), jnp.float32),
          pltpu.VMEM((2, EB), jnp.float32),
          pltpu.VMEM((2, EB), jnp.float32),
          pltpu.VMEM((EB,), jnp.float32),
          pltpu.VMEM((16,), jnp.float32),
          pltpu.VMEM((RPT,), jnp.float32),
          pltpu.SemaphoreType.DMA,
          pltpu.SemaphoreType.DMA,
          pltpu.SemaphoreType.DMA,
          pltpu.SemaphoreType.DMA,
      ],
  )
  def gat_kernel(zr, elr, err, c16r, srcr, dstr, outn, outd,
                 accn, accd, sv, dv, rows, elv, erv, av, cv, stage,
                 gsem, ssem, esem, isem):
    c = lax.axis_index("c")
    s = lax.axis_index("s")
    start, nb = _my_batches(c, s, nbw0, nbw1)
    base = s * RPT
    def zb(i, _):
      for v in range(D // 16):
        rows[0, i, pl.ds(v * 16, 16)] = jnp.zeros((16,), jnp.float32)
      return 0
    lax.fori_loop(0, EB, zb, 0)
    for off, ln in _row_chunks():
      pltpu.sync_copy(rows.at[0, pl.ds(0, ln)],
                      accn.at[pl.ds(base + off, ln)])
    _zero_vec(stage, RPT)
    pltpu.sync_copy(stage, accd.at[pl.ds(base, RPT)])
    plsc.subcore_barrier()
    pltpu.sync_copy(c16r, cv)
    cvec = cv[...]
    pltpu.sync_copy(srcr.at[start], sv.at[0])
    pltpu.sync_copy(dstr.at[start], dv.at[0])
    pltpu.async_copy(srcr.at[start + 1], sv.at[1], isem)
    pltpu.async_copy(dstr.at[start + 1], dv.at[1], isem)
    pltpu.async_copy(elr.at[sv.at[0]], elv.at[0], esem)
    pltpu.async_copy(err.at[dv.at[0]], erv.at[0], esem)
    pltpu.async_copy(zr.at[sv.at[0]], rows.at[0], gsem)
    def eb(b, _):
      cur = lax.rem(b, 2)
      nxt = 1 - cur
      i0 = lax.rem(b, 4)
      i1 = lax.rem(b + 1, 4)
      i2 = lax.rem(b + 2, 4)
      idxd = dv.at[i0]
      pltpu.make_async_copy(elr.at[sv.at[i0]], elv.at[cur], esem).wait()
      pltpu.make_async_copy(err.at[idxd], erv.at[cur], esem).wait()
      for k in range(EB // 16):
        sm = elv[cur, pl.ds(k * 16, 16)] + erv[cur, pl.ds(k * 16, 16)]
        lr = jnp.where(sm > 0, sm, 0.2 * sm)
        av[pl.ds(k * 16, 16)] = jnp.exp(lr - cvec)
      pltpu.sync_copy(av, accd.at[idxd], add=True)
      pltpu.make_async_copy(zr.at[sv.at[i0]], rows.at[cur], gsem).wait()
      @pl.when(b > 0)
      def _():
        pltpu.make_async_copy(
            rows.at[nxt], accn.at[dv.at[lax.rem(b + 3, 4)]], ssem).wait()
      @pl.when(b + 1 < nb)
      def _():
        pltpu.make_async_copy(srcr.at[start + b + 1], sv.at[i1], isem).wait()
        pltpu.make_async_copy(dstr.at[start + b + 1], dv.at[i1], isem).wait()
        pltpu.async_copy(elr.at[sv.at[i1]], elv.at[nxt], esem)
        pltpu.async_copy(err.at[dv.at[i1]], erv.at[nxt], esem)
        pltpu.async_copy(zr.at[sv.at[i1]], rows.at[nxt], gsem)
      @pl.when(b + 2 < nb)
      def _():
        pltpu.async_copy(srcr.at[start + b + 2], sv.at[i2], isem)
        pltpu.async_copy(dstr.at[start + b + 2], dv.at[i2], isem)
      for g in range(EB // 16):
        a16 = av[pl.ds(g * 16, 16)]
        def sb(l, _):
          asp = jnp.take_along_axis(
              a16, lax.broadcast(l, (16,)), axis=0,
              mode="promise_in_bounds")
          row = g * 16 + l
          for v in range(D // 16):
            rows[cur, row, pl.ds(v * 16, 16)] = (
                rows[cur, row, pl.ds(v * 16, 16)] * asp)
          return 0
        lax.fori_loop(0, 16, sb, 0)
      pltpu.async_copy(rows.at[cur], accn.at[idxd], ssem, add=True)
      return 0
    lax.fori_loop(0, nb, eb, 0)
    pltpu.make_async_copy(
        rows.at[lax.rem(nb - 1, 2)], accn.at[dv.at[lax.rem(nb - 1, 4)]],
        ssem).wait()
    plsc.subcore_barrier()
    for off, ln in _row_chunks():
      pltpu.sync_copy(accn.at[pl.ds(base + off, ln)],
                      rows.at[0, pl.ds(0, ln)])
      pltpu.sync_copy(rows.at[0, pl.ds(0, ln)],
                      outn.at[c, pl.ds(base + off, ln)])
    pltpu.sync_copy(accd.at[pl.ds(base, RPT)], stage)
    pltpu.sync_copy(stage, outd.at[c, pl.ds(base, RPT)])

  return gat_kernel


_BLKG = 1024


def _tc_gat_fn(y_ref, w_ref, al_ref, ar_ref, z_ref, el_ref, er_ref, cm_ref):
  i = pl.program_id(0)
  z = jnp.dot(y_ref[...], w_ref[...], preferred_element_type=jnp.float32)
  z_ref[...] = z
  el = jnp.sum(z * al_ref[...], axis=1, keepdims=True)
  er = jnp.sum(z * ar_ref[...], axis=1, keepdims=True)
  el_ref[...] = el
  er_ref[...] = er
  vals = jnp.concatenate(
      [jnp.full((1, 128), jnp.max(el), jnp.float32),
       jnp.full((1, 128), jnp.max(er), jnp.float32)], axis=0)
  @pl.when(i == 0)
  def _():
    cm_ref[...] = vals
  @pl.when(i > 0)
  def _():
    cm_ref[...] = jnp.maximum(cm_ref[...], vals)


def _tc_gat(y, W, al, ar):
  return pl.pallas_call(
      _tc_gat_fn,
      grid=(NPAD // _BLKG,),
      in_specs=[
          pl.BlockSpec((_BLKG, D), lambda i: (i, 0)),
          pl.BlockSpec((D, D), lambda i: (0, 0)),
          pl.BlockSpec((1, D), lambda i: (0, 0)),
          pl.BlockSpec((1, D), lambda i: (0, 0)),
      ],
      out_specs=[
          pl.BlockSpec((_BLKG, D), lambda i: (i, 0)),
          pl.BlockSpec((_BLKG, 1), lambda i: (i, 0)),
          pl.BlockSpec((_BLKG, 1), lambda i: (i, 0)),
          pl.BlockSpec((2, 128), lambda i: (0, 0)),
      ],
      out_shape=[
          jax.ShapeDtypeStruct((NPAD, D), jnp.float32),
          jax.ShapeDtypeStruct((NPAD, 1), jnp.float32),
          jax.ShapeDtypeStruct((NPAD, 1), jnp.float32),
          jax.ShapeDtypeStruct((2, 128), jnp.float32),
      ],
  )(y, W, al.reshape(1, D), ar.reshape(1, D))


def _tc_final_fn(num_ref, den_ref, b_ref, o_ref):
  den = den_ref[...]
  rst = num_ref[...] / jnp.where(den > 0, den, 1.0)
  t = jnp.maximum(rst + b_ref[...], 0.0)
  m = jnp.max(t, axis=1, keepdims=True)
  lse = jnp.log(jnp.sum(jnp.exp(t - m), axis=1, keepdims=True))
  o_ref[...] = t - m - lse


def _tc_final(num, den, bias):
  return pl.pallas_call(
      _tc_final_fn,
      grid=(NPAD // _BLKG,),
      in_specs=[
          pl.BlockSpec((_BLKG, D), lambda i: (i, 0)),
          pl.BlockSpec((_BLKG, 1), lambda i: (i, 0)),
          pl.BlockSpec((1, D), lambda i: (0, 0)),
      ],
      out_specs=pl.BlockSpec((_BLKG, D), lambda i: (i, 0)),
      out_shape=jax.ShapeDtypeStruct((NPAD, D), jnp.float32),
  )(num, den, bias.reshape(1, D))


def kernel(feats, edge_index, W, attn_l, attn_r, bias):
  n, d = feats.shape
  e = edge_index.shape[1]
  ep = -(-e // (NW * EB)) * (NW * EB)
  nbt = ep // EB
  nbw0 = round(nbt * 0.835 / NS)
  nbw1 = nbt // NS - nbw0
  nbw0g = round(nbt * 0.72 / NS)
  nbw1g = nbt // NS - nbw0g

  src = edge_index[0]
  dst = edge_index[1]
  pad = ep - e
  padv = jnp.full((pad,), n, jnp.int32)
  srcp = jnp.concatenate([src, padv]).reshape(nbt, EB)
  dstp = jnp.concatenate([dst, padv]).reshape(nbt, EB)
  feats_p = jnp.zeros((NPAD, d), feats.dtype).at[:n].set(feats)

  degs = _sc_degree(nbt // NS // 2, nbt // NS - nbt // NS // 2)(srcp, dstp)
  out_deg = degs[0, 0] + degs[1, 0]
  in_deg = degs[0, 1] + degs[1, 1]
  norm1 = jax.lax.rsqrt(jnp.clip(in_deg, 1.0))
  norm2 = jax.lax.rsqrt(jnp.clip(out_deg, 1.0))
  s11 = (norm1 * norm1)[:, None]
  s12 = (norm1 * norm2)[:, None]
  n2c = norm2[:, None]

  prop = _sc_prop(nbw0, nbw1)
  raw1 = prop(feats_p * n2c, srcp, dstp)
  in2 = (raw1[0] + raw1[1]) * s11
  raw2 = prop(in2, srcp, dstp)
  r2 = raw2[0] + raw2[1]
  y = feats_p + r2 * n2c
  raw3 = prop(r2 * s12, srcp, dstp)
  r3 = raw3[0] + raw3[1]
  y = y + r3 * n2c
  raw4 = prop(r3 * s12, srcp, dstp)
  r4 = raw4[0] + raw4[1]
  y = (y + r4 * n2c) * 0.25

  z, el, er, cm = _tc_gat(y, W, attn_l, attn_r)
  c16 = jnp.full((16,), cm[0, 0] + cm[1, 0], jnp.float32)
  nump, denp = _sc_gat(nbw0g, nbw1g)(
      z, el.reshape(NPAD), er.reshape(NPAD), c16, srcp, dstp)
  num = nump[0] + nump[1]
  den = (denp[0] + denp[1])[:, None]
  logp = _tc_final(num, den, bias)
  return logp[:n]


# speedup vs baseline: 1.4633x; 1.0001x over previous
import functools

import jax
import jax.numpy as jnp
from jax import lax
from jax.experimental import pallas as pl
from jax.experimental.pallas import tpu as pltpu
from jax.experimental.pallas import tpu_sc as plsc

N = 10000
D = 128
NC = 2
NS = 16
NW = NC * NS
EB = 112
NPAD = 10240
RPT = NPAD // NS

_MESH = dict(core_axis_name="c", subcore_axis_name="s")


def _my_batches(c, s, nbw0, nbw1):
  start = jnp.where(c == 0, s * nbw0, NS * nbw0 + s * nbw1)
  nb = jnp.where(c == 0, nbw0, nbw1)
  return start, nb


def _row_chunks():
  chunks = []
  off = 0
  while off < RPT:
    ln = min(EB, RPT - off)
    chunks.append((off, ln))
    off += ln
  return chunks


def _zero_rows(rows):
  def zb(i, _):
    for v in range(D // 16):
      rows[i, pl.ds(v * 16, 16)] = jnp.zeros((16,), jnp.float32)
    return 0
  lax.fori_loop(0, EB, zb, 0)


def _zero_vec(vec, n):
  def zb(i, _):
    vec[pl.ds(i * 16, 16)] = jnp.zeros((16,), jnp.float32)
    return 0
  lax.fori_loop(0, n // 16, zb, 0)


def _sc_degree(nbw0, nbw1):
  @functools.partial(
      pl.kernel,
      out_type=jax.ShapeDtypeStruct((NC, 2, NPAD), jnp.float32),
      mesh=plsc.VectorSubcoreMesh(**_MESH),
      scratch_types=[
          pltpu.VMEM_SHARED((NPAD,), jnp.float32),
          pltpu.VMEM_SHARED((NPAD,), jnp.float32),
          pltpu.VMEM((4, EB), jnp.int32),
          pltpu.VMEM((4, EB), jnp.int32),
          pltpu.VMEM((EB,), jnp.float32),
          pltpu.VMEM((RPT,), jnp.float32),
          pltpu.SemaphoreType.DMA,
      ],
  )
  def deg_kernel(srcr, dstr, out, acc_s, acc_d, sv, dv, ones, stage, isem):
    c = lax.axis_index("c")
    s = lax.axis_index("s")
    start, nb = _my_batches(c, s, nbw0, nbw1)
    base = s * RPT
    _zero_vec(stage, RPT)
    def ob(i, _):
      ones[pl.ds(i * 16, 16)] = jnp.ones((16,), jnp.float32)
      return 0
    lax.fori_loop(0, EB // 16, ob, 0)
    pltpu.sync_copy(stage, acc_s.at[pl.ds(base, RPT)])
    pltpu.sync_copy(stage, acc_d.at[pl.ds(base, RPT)])
    plsc.subcore_barrier()
    pltpu.sync_copy(srcr.at[start], sv.at[0])
    pltpu.sync_copy(dstr.at[start], dv.at[0])
    pltpu.async_copy(srcr.at[start + 1], sv.at[1], isem)
    pltpu.async_copy(dstr.at[start + 1], dv.at[1], isem)
    def eb(b, _):
      i0 = lax.rem(b, 4)
      i1 = lax.rem(b + 1, 4)
      pltpu.sync_copy(ones, acc_s.at[sv.at[i0]], add=True)
      pltpu.sync_copy(ones, acc_d.at[dv.at[i0]], add=True)
      @pl.when(b + 1 < nb)
      def _():
        pltpu.make_async_copy(srcr.at[start + b + 1], sv.at[i1], isem).wait()
        pltpu.make_async_copy(dstr.at[start + b + 1], dv.at[i1], isem).wait()
      @pl.when(b + 2 < nb)
      def _():
        i2 = lax.rem(b + 2, 4)
        pltpu.async_copy(srcr.at[start + b + 2], sv.at[i2], isem)
        pltpu.async_copy(dstr.at[start + b + 2], dv.at[i2], isem)
      return 0
    lax.fori_loop(0, nb, eb, 0)
    plsc.subcore_barrier()
    pltpu.sync_copy(acc_s.at[pl.ds(base, RPT)], stage)
    pltpu.sync_copy(stage, out.at[c, 0, pl.ds(base, RPT)])
    pltpu.sync_copy(acc_d.at[pl.ds(base, RPT)], stage)
    pltpu.sync_copy(stage, out.at[c, 1, pl.ds(base, RPT)])

  return deg_kernel


def _sc_prop(nbw0, nbw1):
  @functools.partial(
      pl.kernel,
      out_type=jax.ShapeDtypeStruct((NC, NPAD, D), jnp.float32),
      mesh=plsc.VectorSubcoreMesh(**_MESH),
      scratch_types=[
          pltpu.VMEM_SHARED((NPAD, D), jnp.float32),
          pltpu.VMEM((4, EB), jnp.int32),
          pltpu.VMEM((4, EB), jnp.int32),
          pltpu.VMEM((3, EB, D), jnp.float32),
          pltpu.SemaphoreType.DMA,
          pltpu.SemaphoreType.DMA,
          pltpu.SemaphoreType.DMA,
          pltpu.SemaphoreType.DMA,
      ],
  )
  def prop_kernel(xr, srcr, dstr, out, acc, sv, dv, rows, gsa, gsb, ssem,
                  isem):
    c = lax.axis_index("c")
    s = lax.axis_index("s")
    start, nb = _my_batches(c, s, nbw0, nbw1)
    base = s * RPT
    def zb(i, _):
      for v in range(D // 16):
        rows[0, i, pl.ds(v * 16, 16)] = jnp.zeros((16,), jnp.float32)
      return 0
    lax.fori_loop(0, EB, zb, 0)
    for off, ln in _row_chunks():
      pltpu.sync_copy(rows.at[0, pl.ds(0, ln)],
                      acc.at[pl.ds(base + off, ln)])
    plsc.subcore_barrier()
    pltpu.sync_copy(srcr.at[start], sv.at[0])
    pltpu.sync_copy(dstr.at[start], dv.at[0])
    pltpu.sync_copy(srcr.at[start + 1], sv.at[1])
    pltpu.sync_copy(dstr.at[start + 1], dv.at[1])
    pltpu.async_copy(srcr.at[start + 2], sv.at[2], isem)
    pltpu.async_copy(dstr.at[start + 2], dv.at[2], isem)
    pltpu.async_copy(xr.at[sv.at[0]], rows.at[0], gsa)
    pltpu.async_copy(xr.at[sv.at[1]], rows.at[1], gsb)
    def eb(b, _):
      cur = lax.rem(b, 3)
      pre = lax.rem(b + 2, 3)
      i0 = lax.rem(b, 4)
      i2 = lax.rem(b + 2, 4)
      i3 = lax.rem(b + 3, 4)
      even = lax.rem(b, 2) == 0
      @pl.when(even)
      def _():
        pltpu.make_async_copy(xr.at[sv.at[i0]], rows.at[cur], gsa).wait()
      @pl.when(jnp.logical_not(even))
      def _():
        pltpu.make_async_copy(xr.at[sv.at[i0]], rows.at[cur], gsb).wait()
      @pl.when(b > 0)
      def _():
        pltpu.make_async_copy(
            rows.at[pre], acc.at[dv.at[lax.rem(b + 3, 4)]], ssem).wait()
      @pl.when(b + 2 < nb)
      def _():
        pltpu.make_async_copy(srcr.at[start + b + 2], sv.at[i2], isem).wait()
        pltpu.make_async_copy(dstr.at[start + b + 2], dv.at[i2], isem).wait()
        @pl.when(even)
        def _():
          pltpu.async_copy(xr.at[sv.at[i2]], rows.at[pre], gsa)
        @pl.when(jnp.logical_not(even))
        def _():
          pltpu.async_copy(xr.at[sv.at[i2]], rows.at[pre], gsb)
      @pl.when(b + 3 < nb)
      def _():
        pltpu.async_copy(srcr.at[start + b + 3], sv.at[i3], isem)
        pltpu.async_copy(dstr.at[start + b + 3], dv.at[i3], isem)
      pltpu.async_copy(rows.at[cur], acc.at[dv.at[i0]], ssem, add=True)
      return 0
    lax.fori_loop(0, nb, eb, 0)
    pltpu.make_async_copy(
        rows.at[lax.rem(nb - 1, 3)], acc.at[dv.at[lax.rem(nb - 1, 4)]],
        ssem).wait()
    plsc.subcore_barrier()
    for off, ln in _row_chunks():
      pltpu.sync_copy(acc.at[pl.ds(base + off, ln)],
                      rows.at[0, pl.ds(0, ln)])
      pltpu.sync_copy(rows.at[0, pl.ds(0, ln)],
                      out.at[c, pl.ds(base + off, ln)])

  return prop_kernel


def _sc_gat(nbw0, nbw1):
  @functools.partial(
      pl.kernel,
      out_type=(
          jax.ShapeDtypeStruct((NC, NPAD, D), jnp.float32),
          jax.ShapeDtypeStruct((NC, NPAD), jnp.float32),
      ),
      mesh=plsc.VectorSubcoreMesh(**_MESH),
      scratch_types=[
          pltpu.VMEM_SHARED((NPAD, D), jnp.float32),
          pltpu.VMEM_SHARED((NPAD,), jnp.float32),
          pltpu.VMEM((4, EB), jnp.int32),
          pltpu.VMEM((4, EB), jnp.int32),
          pltpu.VMEM((2, EB, D), jnp.float32),
          pltpu.VMEM((2, EB), jnp.float32),
          pltpu.VMEM((2, EB), jnp.float32),
          pltpu.VMEM((EB,), jnp.float32),
          pltpu.VMEM((16,), jnp.float32),
          pltpu.VMEM((RPT,), jnp.float32),
          pltpu.SemaphoreType.DMA,
          pltpu.SemaphoreType.DMA,
          pltpu.SemaphoreType.DMA,
          pltpu.SemaphoreType.DMA,
      ],
  )
  def gat_kernel(zr, elr, err, c16r, srcr, dstr, outn, outd,
                 accn, accd, sv, dv, rows, elv, erv, av, cv, stage,
                 gsem, ssem, esem, isem):
    c = lax.axis_index("c")
    s = lax.axis_index("s")
    start, nb = _my_batches(c, s, nbw0, nbw1)
    base = s * RPT
    def zb(i, _):
      for v in range(D // 16):
        rows[0, i, pl.ds(v * 16, 16)] = jnp.zeros((16,), jnp.float32)
      return 0
    lax.fori_loop(0, EB, zb, 0)
    for off, ln in _row_chunks():
      pltpu.sync_copy(rows.at[0, pl.ds(0, ln)],
                      accn.at[pl.ds(base + off, ln)])
    _zero_vec(stage, RPT)
    pltpu.sync_copy(stage, accd.at[pl.ds(base, RPT)])
    plsc.subcore_barrier()
    pltpu.sync_copy(c16r, cv)
    cvec = cv[...]
    pltpu.sync_copy(srcr.at[start], sv.at[0])
    pltpu.sync_copy(dstr.at[start], dv.at[0])
    pltpu.async_copy(srcr.at[start + 1], sv.at[1], isem)
    pltpu.async_copy(dstr.at[start + 1], dv.at[1], isem)
    pltpu.async_copy(elr.at[sv.at[0]], elv.at[0], esem)
    pltpu.async_copy(err.at[dv.at[0]], erv.at[0], esem)
    pltpu.async_copy(zr.at[sv.at[0]], rows.at[0], gsem)
    def eb(b, _):
      cur = lax.rem(b, 2)
      nxt = 1 - cur
      i0 = lax.rem(b, 4)
      i1 = lax.rem(b + 1, 4)
      i2 = lax.rem(b + 2, 4)
      idxd = dv.at[i0]
      pltpu.make_async_copy(elr.at[sv.at[i0]], elv.at[cur], esem).wait()
      pltpu.make_async_copy(err.at[idxd], erv.at[cur], esem).wait()
      for k in range(EB // 16):
        sm = elv[cur, pl.ds(k * 16, 16)] + erv[cur, pl.ds(k * 16, 16)]
        lr = jnp.where(sm > 0, sm, 0.2 * sm)
        av[pl.ds(k * 16, 16)] = jnp.exp(lr - cvec)
      pltpu.sync_copy(av, accd.at[idxd], add=True)
      pltpu.make_async_copy(zr.at[sv.at[i0]], rows.at[cur], gsem).wait()
      @pl.when(b > 0)
      def _():
        pltpu.make_async_copy(
            rows.at[nxt], accn.at[dv.at[lax.rem(b + 3, 4)]], ssem).wait()
      @pl.when(b + 1 < nb)
      def _():
        pltpu.make_async_copy(srcr.at[start + b + 1], sv.at[i1], isem).wait()
        pltpu.make_async_copy(dstr.at[start + b + 1], dv.at[i1], isem).wait()
        pltpu.async_copy(elr.at[sv.at[i1]], elv.at[nxt], esem)
        pltpu.async_copy(err.at[dv.at[i1]], erv.at[nxt], esem)
        pltpu.async_copy(zr.at[sv.at[i1]], rows.at[nxt], gsem)
      @pl.when(b + 2 < nb)
      def _():
        pltpu.async_copy(srcr.at[start + b + 2], sv.at[i2], isem)
        pltpu.async_copy(dstr.at[start + b + 2], dv.at[i2], isem)
      for g in range(EB // 16):
        a16 = av[pl.ds(g * 16, 16)]
        def sb(l, _):
          asp = jnp.take_along_axis(
              a16, lax.broadcast(l, (16,)), axis=0,
              mode="promise_in_bounds")
          asp2 = jnp.take_along_axis(
              a16, lax.broadcast(l + 1, (16,)), axis=0,
              mode="promise_in_bounds")
          row = g * 16 + l
          for v in range(D // 16):
            rows[cur, row, pl.ds(v * 16, 16)] = (
                rows[cur, row, pl.ds(v * 16, 16)] * asp)
          for v in range(D // 16):
            rows[cur, row + 1, pl.ds(v * 16, 16)] = (
                rows[cur, row + 1, pl.ds(v * 16, 16)] * asp2)
          return 0
        lax.fori_loop(0, 8, lambda j, o: sb(2 * j, o), 0)
      pltpu.async_copy(rows.at[cur], accn.at[idxd], ssem, add=True)
      return 0
    lax.fori_loop(0, nb, eb, 0)
    pltpu.make_async_copy(
        rows.at[lax.rem(nb - 1, 2)], accn.at[dv.at[lax.rem(nb - 1, 4)]],
        ssem).wait()
    plsc.subcore_barrier()
    for off, ln in _row_chunks():
      pltpu.sync_copy(accn.at[pl.ds(base + off, ln)],
                      rows.at[0, pl.ds(0, ln)])
      pltpu.sync_copy(rows.at[0, pl.ds(0, ln)],
                      outn.at[c, pl.ds(base + off, ln)])
    pltpu.sync_copy(accd.at[pl.ds(base, RPT)], stage)
    pltpu.sync_copy(stage, outd.at[c, pl.ds(base, RPT)])

  return gat_kernel


_BLKG = 1024


def _tc_gat_fn(y_ref, w_ref, al_ref, ar_ref, z_ref, el_ref, er_ref, cm_ref):
  i = pl.program_id(0)
  z = jnp.dot(y_ref[...], w_ref[...], preferred_element_type=jnp.float32)
  z_ref[...] = z
  el = jnp.sum(z * al_ref[...], axis=1, keepdims=True)
  er = jnp.sum(z * ar_ref[...], axis=1, keepdims=True)
  el_ref[...] = el
  er_ref[...] = er
  vals = jnp.concatenate(
      [jnp.full((1, 128), jnp.max(el), jnp.float32),
       jnp.full((1, 128), jnp.max(er), jnp.float32)], axis=0)
  @pl.when(i == 0)
  def _():
    cm_ref[...] = vals
  @pl.when(i > 0)
  def _():
    cm_ref[...] = jnp.maximum(cm_ref[...], vals)


def _tc_gat(y, W, al, ar):
  return pl.pallas_call(
      _tc_gat_fn,
      grid=(NPAD // _BLKG,),
      in_specs=[
          pl.BlockSpec((_BLKG, D), lambda i: (i, 0)),
          pl.BlockSpec((D, D), lambda i: (0, 0)),
          pl.BlockSpec((1, D), lambda i: (0, 0)),
          pl.BlockSpec((1, D), lambda i: (0, 0)),
      ],
      out_specs=[
          pl.BlockSpec((_BLKG, D), lambda i: (i, 0)),
          pl.BlockSpec((_BLKG, 1), lambda i: (i, 0)),
          pl.BlockSpec((_BLKG, 1), lambda i: (i, 0)),
          pl.BlockSpec((2, 128), lambda i: (0, 0)),
      ],
      out_shape=[
          jax.ShapeDtypeStruct((NPAD, D), jnp.float32),
          jax.ShapeDtypeStruct((NPAD, 1), jnp.float32),
          jax.ShapeDtypeStruct((NPAD, 1), jnp.float32),
          jax.ShapeDtypeStruct((2, 128), jnp.float32),
      ],
  )(y, W, al.reshape(1, D), ar.reshape(1, D))


def _tc_final_fn(num_ref, den_ref, b_ref, o_ref):
  den = den_ref[...]
  rst = num_ref[...] / jnp.where(den > 0, den, 1.0)
  t = jnp.maximum(rst + b_ref[...], 0.0)
  m = jnp.max(t, axis=1, keepdims=True)
  lse = jnp.log(jnp.sum(jnp.exp(t - m), axis=1, keepdims=True))
  o_ref[...] = t - m - lse


def _tc_final(num, den, bias):
  return pl.pallas_call(
      _tc_final_fn,
      grid=(NPAD // _BLKG,),
      in_specs=[
          pl.BlockSpec((_BLKG, D), lambda i: (i, 0)),
          pl.BlockSpec((_BLKG, 1), lambda i: (i, 0)),
          pl.BlockSpec((1, D), lambda i: (0, 0)),
      ],
      out_specs=pl.BlockSpec((_BLKG, D), lambda i: (i, 0)),
      out_shape=jax.ShapeDtypeStruct((NPAD, D), jnp.float32),
  )(num, den, bias.reshape(1, D))


def kernel(feats, edge_index, W, attn_l, attn_r, bias):
  n, d = feats.shape
  e = edge_index.shape[1]
  ep = -(-e // (NW * EB)) * (NW * EB)
  nbt = ep // EB
  nbw0 = round(nbt * 0.835 / NS)
  nbw1 = nbt // NS - nbw0
  nbw0g = round(nbt * 0.72 / NS)
  nbw1g = nbt // NS - nbw0g

  src = edge_index[0]
  dst = edge_index[1]
  pad = ep - e
  padv = jnp.full((pad,), n, jnp.int32)
  srcp = jnp.concatenate([src, padv]).reshape(nbt, EB)
  dstp = jnp.concatenate([dst, padv]).reshape(nbt, EB)
  feats_p = jnp.zeros((NPAD, d), feats.dtype).at[:n].set(feats)

  degs = _sc_degree(nbt // NS // 2, nbt // NS - nbt // NS // 2)(srcp, dstp)
  out_deg = degs[0, 0] + degs[1, 0]
  in_deg = degs[0, 1] + degs[1, 1]
  norm1 = jax.lax.rsqrt(jnp.clip(in_deg, 1.0))
  norm2 = jax.lax.rsqrt(jnp.clip(out_deg, 1.0))
  s11 = (norm1 * norm1)[:, None]
  s12 = (norm1 * norm2)[:, None]
  n2c = norm2[:, None]

  prop = _sc_prop(nbw0, nbw1)
  raw1 = prop(feats_p * n2c, srcp, dstp)
  in2 = (raw1[0] + raw1[1]) * s11
  raw2 = prop(in2, srcp, dstp)
  r2 = raw2[0] + raw2[1]
  y = feats_p + r2 * n2c
  raw3 = prop(r2 * s12, srcp, dstp)
  r3 = raw3[0] + raw3[1]
  y = y + r3 * n2c
  raw4 = prop(r3 * s12, srcp, dstp)
  r4 = raw4[0] + raw4[1]
  y = (y + r4 * n2c) * 0.25

  z, el, er, cm = _tc_gat(y, W, attn_l, attn_r)
  c16 = jnp.full((16,), cm[0, 0] + cm[1, 0], jnp.float32)
  nump, denp = _sc_gat(nbw0g, nbw1g)(
      z, el.reshape(NPAD), er.reshape(NPAD), c16, srcp, dstp)
  num = nump[0] + nump[1]
  den = (denp[0] + denp[1])[:, None]
  logp = _tc_final(num, den, bias)
  return logp[:n]


# speedup vs baseline: 1.4635x; 1.0001x over previous
import functools

import jax
import jax.numpy as jnp
from jax import lax
from jax.experimental import pallas as pl
from jax.experimental.pallas import tpu as pltpu
from jax.experimental.pallas import tpu_sc as plsc

N = 10000
D = 128
NC = 2
NS = 16
NW = NC * NS
EB = 112
NPAD = 10240
RPT = NPAD // NS

_MESH = dict(core_axis_name="c", subcore_axis_name="s")


def _my_batches(c, s, nbw0, nbw1):
  start = jnp.where(c == 0, s * nbw0, NS * nbw0 + s * nbw1)
  nb = jnp.where(c == 0, nbw0, nbw1)
  return start, nb


def _row_chunks():
  chunks = []
  off = 0
  while off < RPT:
    ln = min(EB, RPT - off)
    chunks.append((off, ln))
    off += ln
  return chunks


def _zero_rows(rows):
  def zb(i, _):
    for v in range(D // 16):
      rows[i, pl.ds(v * 16, 16)] = jnp.zeros((16,), jnp.float32)
    return 0
  lax.fori_loop(0, EB, zb, 0)


def _zero_vec(vec, n):
  def zb(i, _):
    vec[pl.ds(i * 16, 16)] = jnp.zeros((16,), jnp.float32)
    return 0
  lax.fori_loop(0, n // 16, zb, 0)


def _sc_degree(nbw0, nbw1):
  @functools.partial(
      pl.kernel,
      out_type=jax.ShapeDtypeStruct((NC, 2, NPAD), jnp.float32),
      mesh=plsc.VectorSubcoreMesh(**_MESH),
      scratch_types=[
          pltpu.VMEM_SHARED((NPAD,), jnp.float32),
          pltpu.VMEM_SHARED((NPAD,), jnp.float32),
          pltpu.VMEM((4, EB), jnp.int32),
          pltpu.VMEM((4, EB), jnp.int32),
          pltpu.VMEM((EB,), jnp.float32),
          pltpu.VMEM((RPT,), jnp.float32),
          pltpu.SemaphoreType.DMA,
          pltpu.SemaphoreType.DMA,
          pltpu.SemaphoreType.DMA,
      ],
  )
  def deg_kernel(srcr, dstr, out, acc_s, acc_d, sv, dv, ones, stage, isem,
                 dsa, dsb):
    c = lax.axis_index("c")
    s = lax.axis_index("s")
    start, nb = _my_batches(c, s, nbw0, nbw1)
    base = s * RPT
    _zero_vec(stage, RPT)
    def ob(i, _):
      ones[pl.ds(i * 16, 16)] = jnp.ones((16,), jnp.float32)
      return 0
    lax.fori_loop(0, EB // 16, ob, 0)
    pltpu.sync_copy(stage, acc_s.at[pl.ds(base, RPT)])
    pltpu.sync_copy(stage, acc_d.at[pl.ds(base, RPT)])
    plsc.subcore_barrier()
    pltpu.sync_copy(srcr.at[start], sv.at[0])
    pltpu.sync_copy(dstr.at[start], dv.at[0])
    pltpu.async_copy(srcr.at[start + 1], sv.at[1], isem)
    pltpu.async_copy(dstr.at[start + 1], dv.at[1], isem)
    def eb(b, _):
      i0 = lax.rem(b, 4)
      i1 = lax.rem(b + 1, 4)
      @pl.when(b > 0)
      def _():
        ip = lax.rem(b + 3, 4)
        pltpu.make_async_copy(ones, acc_s.at[sv.at[ip]], dsa).wait()
        pltpu.make_async_copy(ones, acc_d.at[dv.at[ip]], dsb).wait()
      pltpu.async_copy(ones, acc_s.at[sv.at[i0]], dsa, add=True)
      pltpu.async_copy(ones, acc_d.at[dv.at[i0]], dsb, add=True)
      @pl.when(b + 1 < nb)
      def _():
        pltpu.make_async_copy(srcr.at[start + b + 1], sv.at[i1], isem).wait()
        pltpu.make_async_copy(dstr.at[start + b + 1], dv.at[i1], isem).wait()
      @pl.when(b + 2 < nb)
      def _():
        i2 = lax.rem(b + 2, 4)
        pltpu.async_copy(srcr.at[start + b + 2], sv.at[i2], isem)
        pltpu.async_copy(dstr.at[start + b + 2], dv.at[i2], isem)
      return 0
    lax.fori_loop(0, nb, eb, 0)
    pltpu.make_async_copy(
        ones, acc_s.at[sv.at[lax.rem(nb - 1, 4)]], dsa).wait()
    pltpu.make_async_copy(
        ones, acc_d.at[dv.at[lax.rem(nb - 1, 4)]], dsb).wait()
    plsc.subcore_barrier()
    pltpu.sync_copy(acc_s.at[pl.ds(base, RPT)], stage)
    pltpu.sync_copy(stage, out.at[c, 0, pl.ds(base, RPT)])
    pltpu.sync_copy(acc_d.at[pl.ds(base, RPT)], stage)
    pltpu.sync_copy(stage, out.at[c, 1, pl.ds(base, RPT)])

  return deg_kernel


def _sc_prop(nbw0, nbw1):
  @functools.partial(
      pl.kernel,
      out_type=jax.ShapeDtypeStruct((NC, NPAD, D), jnp.float32),
      mesh=plsc.VectorSubcoreMesh(**_MESH),
      scratch_types=[
          pltpu.VMEM_SHARED((NPAD, D), jnp.float32),
          pltpu.VMEM((4, EB), jnp.int32),
          pltpu.VMEM((4, EB), jnp.int32),
          pltpu.VMEM((3, EB, D), jnp.float32),
          pltpu.SemaphoreType.DMA,
          pltpu.SemaphoreType.DMA,
          pltpu.SemaphoreType.DMA,
          pltpu.SemaphoreType.DMA,
      ],
  )
  def prop_kernel(xr, srcr, dstr, out, acc, sv, dv, rows, gsa, gsb, ssem,
                  isem):
    c = lax.axis_index("c")
    s = lax.axis_index("s")
    start, nb = _my_batches(c, s, nbw0, nbw1)
    base = s * RPT
    def zb(i, _):
      for v in range(D // 16):
        rows[0, i, pl.ds(v * 16, 16)] = jnp.zeros((16,), jnp.float32)
      return 0
    lax.fori_loop(0, EB, zb, 0)
    for off, ln in _row_chunks():
      pltpu.sync_copy(rows.at[0, pl.ds(0, ln)],
                      acc.at[pl.ds(base + off, ln)])
    plsc.subcore_barrier()
    pltpu.sync_copy(srcr.at[start], sv.at[0])
    pltpu.sync_copy(dstr.at[start], dv.at[0])
    pltpu.sync_copy(srcr.at[start + 1], sv.at[1])
    pltpu.sync_copy(dstr.at[start + 1], dv.at[1])
    pltpu.async_copy(srcr.at[start + 2], sv.at[2], isem)
    pltpu.async_copy(dstr.at[start + 2], dv.at[2], isem)
    pltpu.async_copy(xr.at[sv.at[0]], rows.at[0], gsa)
    pltpu.async_copy(xr.at[sv.at[1]], rows.at[1], gsb)
    def eb(b, _):
      cur = lax.rem(b, 3)
      pre = lax.rem(b + 2, 3)
      i0 = lax.rem(b, 4)
      i2 = lax.rem(b + 2, 4)
      i3 = lax.rem(b + 3, 4)
      even = lax.rem(b, 2) == 0
      @pl.when(even)
      def _():
        pltpu.make_async_copy(xr.at[sv.at[i0]], rows.at[cur], gsa).wait()
      @pl.when(jnp.logical_not(even))
      def _():
        pltpu.make_async_copy(xr.at[sv.at[i0]], rows.at[cur], gsb).wait()
      @pl.when(b > 0)
      def _():
        pltpu.make_async_copy(
            rows.at[pre], acc.at[dv.at[lax.rem(b + 3, 4)]], ssem).wait()
      @pl.when(b + 2 < nb)
      def _():
        pltpu.make_async_copy(srcr.at[start + b + 2], sv.at[i2], isem).wait()
        pltpu.make_async_copy(dstr.at[start + b + 2], dv.at[i2], isem).wait()
        @pl.when(even)
        def _():
          pltpu.async_copy(xr.at[sv.at[i2]], rows.at[pre], gsa)
        @pl.when(jnp.logical_not(even))
        def _():
          pltpu.async_copy(xr.at[sv.at[i2]], rows.at[pre], gsb)
      @pl.when(b + 3 < nb)
      def _():
        pltpu.async_copy(srcr.at[start + b + 3], sv.at[i3], isem)
        pltpu.async_copy(dstr.at[start + b + 3], dv.at[i3], isem)
      pltpu.async_copy(rows.at[cur], acc.at[dv.at[i0]], ssem, add=True)
      return 0
    lax.fori_loop(0, nb, eb, 0)
    pltpu.make_async_copy(
        rows.at[lax.rem(nb - 1, 3)], acc.at[dv.at[lax.rem(nb - 1, 4)]],
        ssem).wait()
    plsc.subcore_barrier()
    for off, ln in _row_chunks():
      pltpu.sync_copy(acc.at[pl.ds(base + off, ln)],
                      rows.at[0, pl.ds(0, ln)])
      pltpu.sync_copy(rows.at[0, pl.ds(0, ln)],
                      out.at[c, pl.ds(base + off, ln)])

  return prop_kernel


def _sc_gat(nbw0, nbw1):
  @functools.partial(
      pl.kernel,
      out_type=(
          jax.ShapeDtypeStruct((NC, NPAD, D), jnp.float32),
          jax.ShapeDtypeStruct((NC, NPAD), jnp.float32),
      ),
      mesh=plsc.VectorSubcoreMesh(**_MESH),
      scratch_types=[
          pltpu.VMEM_SHARED((NPAD, D), jnp.float32),
          pltpu.VMEM_SHARED((NPAD,), jnp.float32),
          pltpu.VMEM((4, EB), jnp.int32),
          pltpu.VMEM((4, EB), jnp.int32),
          pltpu.VMEM((2, EB, D), jnp.float32),
          pltpu.VMEM((2, EB), jnp.float32),
          pltpu.VMEM((2, EB), jnp.float32),
          pltpu.VMEM((EB,), jnp.float32),
          pltpu.VMEM((16,), jnp.float32),
          pltpu.VMEM((RPT,), jnp.float32),
          pltpu.SemaphoreType.DMA,
          pltpu.SemaphoreType.DMA,
          pltpu.SemaphoreType.DMA,
          pltpu.SemaphoreType.DMA,
      ],
  )
  def gat_kernel(zr, elr, err, c16r, srcr, dstr, outn, outd,
                 accn, accd, sv, dv, rows, elv, erv, av, cv, stage,
                 gsem, ssem, esem, isem):
    c = lax.axis_index("c")
    s = lax.axis_index("s")
    start, nb = _my_batches(c, s, nbw0, nbw1)
    base = s * RPT
    def zb(i, _):
      for v in range(D // 16):
        rows[0, i, pl.ds(v * 16, 16)] = jnp.zeros((16,), jnp.float32)
      return 0
    lax.fori_loop(0, EB, zb, 0)
    for off, ln in _row_chunks():
      pltpu.sync_copy(rows.at[0, pl.ds(0, ln)],
                      accn.at[pl.ds(base + off, ln)])
    _zero_vec(stage, RPT)
    pltpu.sync_copy(stage, accd.at[pl.ds(base, RPT)])
    plsc.subcore_barrier()
    pltpu.sync_copy(c16r, cv)
    cvec = cv[...]
    pltpu.sync_copy(srcr.at[start], sv.at[0])
    pltpu.sync_copy(dstr.at[start], dv.at[0])
    pltpu.async_copy(srcr.at[start + 1], sv.at[1], isem)
    pltpu.async_copy(dstr.at[start + 1], dv.at[1], isem)
    pltpu.async_copy(elr.at[sv.at[0]], elv.at[0], esem)
    pltpu.async_copy(err.at[dv.at[0]], erv.at[0], esem)
    pltpu.async_copy(zr.at[sv.at[0]], rows.at[0], gsem)
    def eb(b, _):
      cur = lax.rem(b, 2)
      nxt = 1 - cur
      i0 = lax.rem(b, 4)
      i1 = lax.rem(b + 1, 4)
      i2 = lax.rem(b + 2, 4)
      idxd = dv.at[i0]
      pltpu.make_async_copy(elr.at[sv.at[i0]], elv.at[cur], esem).wait()
      pltpu.make_async_copy(err.at[idxd], erv.at[cur], esem).wait()
      for k in range(EB // 16):
        sm = elv[cur, pl.ds(k * 16, 16)] + erv[cur, pl.ds(k * 16, 16)]
        lr = jnp.where(sm > 0, sm, 0.2 * sm)
        av[pl.ds(k * 16, 16)] = jnp.exp(lr - cvec)
      pltpu.sync_copy(av, accd.at[idxd], add=True)
      pltpu.make_async_copy(zr.at[sv.at[i0]], rows.at[cur], gsem).wait()
      @pl.when(b > 0)
      def _():
        pltpu.make_async_copy(
            rows.at[nxt], accn.at[dv.at[lax.rem(b + 3, 4)]], ssem).wait()
      @pl.when(b + 1 < nb)
      def _():
        pltpu.make_async_copy(srcr.at[start + b + 1], sv.at[i1], isem).wait()
        pltpu.make_async_copy(dstr.at[start + b + 1], dv.at[i1], isem).wait()
        pltpu.async_copy(elr.at[sv.at[i1]], elv.at[nxt], esem)
        pltpu.async_copy(err.at[dv.at[i1]], erv.at[nxt], esem)
        pltpu.async_copy(zr.at[sv.at[i1]], rows.at[nxt], gsem)
      @pl.when(b + 2 < nb)
      def _():
        pltpu.async_copy(srcr.at[start + b + 2], sv.at[i2], isem)
        pltpu.async_copy(dstr.at[start + b + 2], dv.at[i2], isem)
      for g in range(EB // 16):
        a16 = av[pl.ds(g * 16, 16)]
        def sb(l, _):
          asp = jnp.take_along_axis(
              a16, lax.broadcast(l, (16,)), axis=0,
              mode="promise_in_bounds")
          asp2 = jnp.take_along_axis(
              a16, lax.broadcast(l + 1, (16,)), axis=0,
              mode="promise_in_bounds")
          row = g * 16 + l
          for v in range(D // 16):
            rows[cur, row, pl.ds(v * 16, 16)] = (
                rows[cur, row, pl.ds(v * 16, 16)] * asp)
          for v in range(D // 16):
            rows[cur, row + 1, pl.ds(v * 16, 16)] = (
                rows[cur, row + 1, pl.ds(v * 16, 16)] * asp2)
          return 0
        lax.fori_loop(0, 8, lambda j, o: sb(2 * j, o), 0)
      pltpu.async_copy(rows.at[cur], accn.at[idxd], ssem, add=True)
      return 0
    lax.fori_loop(0, nb, eb, 0)
    pltpu.make_async_copy(
        rows.at[lax.rem(nb - 1, 2)], accn.at[dv.at[lax.rem(nb - 1, 4)]],
        ssem).wait()
    plsc.subcore_barrier()
    for off, ln in _row_chunks():
      pltpu.sync_copy(accn.at[pl.ds(base + off, ln)],
                      rows.at[0, pl.ds(0, ln)])
      pltpu.sync_copy(rows.at[0, pl.ds(0, ln)],
                      outn.at[c, pl.ds(base + off, ln)])
    pltpu.sync_copy(accd.at[pl.ds(base, RPT)], stage)
    pltpu.sync_copy(stage, outd.at[c, pl.ds(base, RPT)])

  return gat_kernel


_BLKG = 1024


def _tc_gat_fn(y_ref, w_ref, al_ref, ar_ref, z_ref, el_ref, er_ref, cm_ref):
  i = pl.program_id(0)
  z = jnp.dot(y_ref[...], w_ref[...], preferred_element_type=jnp.float32)
  z_ref[...] = z
  el = jnp.sum(z * al_ref[...], axis=1, keepdims=True)
  er = jnp.sum(z * ar_ref[...], axis=1, keepdims=True)
  el_ref[...] = el
  er_ref[...] = er
  vals = jnp.concatenate(
      [jnp.full((1, 128), jnp.max(el), jnp.float32),
       jnp.full((1, 128), jnp.max(er), jnp.float32)], axis=0)
  @pl.when(i == 0)
  def _():
    cm_ref[...] = vals
  @pl.when(i > 0)
  def _():
    cm_ref[...] = jnp.maximum(cm_ref[...], vals)


def _tc_gat(y, W, al, ar):
  return pl.pallas_call(
      _tc_gat_fn,
      grid=(NPAD // _BLKG,),
      in_specs=[
          pl.BlockSpec((_BLKG, D), lambda i: (i, 0)),
          pl.BlockSpec((D, D), lambda i: (0, 0)),
          pl.BlockSpec((1, D), lambda i: (0, 0)),
          pl.BlockSpec((1, D), lambda i: (0, 0)),
      ],
      out_specs=[
          pl.BlockSpec((_BLKG, D), lambda i: (i, 0)),
          pl.BlockSpec((_BLKG, 1), lambda i: (i, 0)),
          pl.BlockSpec((_BLKG, 1), lambda i: (i, 0)),
          pl.BlockSpec((2, 128), lambda i: (0, 0)),
      ],
      out_shape=[
          jax.ShapeDtypeStruct((NPAD, D), jnp.float32),
          jax.ShapeDtypeStruct((NPAD, 1), jnp.float32),
          jax.ShapeDtypeStruct((NPAD, 1), jnp.float32),
          jax.ShapeDtypeStruct((2, 128), jnp.float32),
      ],
  )(y, W, al.reshape(1, D), ar.reshape(1, D))


def _tc_final_fn(num_ref, den_ref, b_ref, o_ref):
  den = den_ref[...]
  rst = num_ref[...] / jnp.where(den > 0, den, 1.0)
  t = jnp.maximum(rst + b_ref[...], 0.0)
  m = jnp.max(t, axis=1, keepdims=True)
  lse = jnp.log(jnp.sum(jnp.exp(t - m), axis=1, keepdims=True))
  o_ref[...] = t - m - lse


def _tc_final(num, den, bias):
  return pl.pallas_call(
      _tc_final_fn,
      grid=(NPAD // _BLKG,),
      in_specs=[
          pl.BlockSpec((_BLKG, D), lambda i: (i, 0)),
          pl.BlockSpec((_BLKG, 1), lambda i: (i, 0)),
          pl.BlockSpec((1, D), lambda i: (0, 0)),
      ],
      out_specs=pl.BlockSpec((_BLKG, D), lambda i: (i, 0)),
      out_shape=jax.ShapeDtypeStruct((NPAD, D), jnp.float32),
  )(num, den, bias.reshape(1, D))


def kernel(feats, edge_index, W, attn_l, attn_r, bias):
  n, d = feats.shape
  e = edge_index.shape[1]
  ep = -(-e // (NW * EB)) * (NW * EB)
  nbt = ep // EB
  nbw0 = round(nbt * 0.835 / NS)
  nbw1 = nbt // NS - nbw0
  nbw0g = round(nbt * 0.72 / NS)
  nbw1g = nbt // NS - nbw0g

  src = edge_index[0]
  dst = edge_index[1]
  pad = ep - e
  padv = jnp.full((pad,), n, jnp.int32)
  srcp = jnp.concatenate([src, padv]).reshape(nbt, EB)
  dstp = jnp.concatenate([dst, padv]).reshape(nbt, EB)
  feats_p = jnp.zeros((NPAD, d), feats.dtype).at[:n].set(feats)

  degs = _sc_degree(nbt // NS // 2, nbt // NS - nbt // NS // 2)(srcp, dstp)
  out_deg = degs[0, 0] + degs[1, 0]
  in_deg = degs[0, 1] + degs[1, 1]
  norm1 = jax.lax.rsqrt(jnp.clip(in_deg, 1.0))
  norm2 = jax.lax.rsqrt(jnp.clip(out_deg, 1.0))
  s11 = (norm1 * norm1)[:, None]
  s12 = (norm1 * norm2)[:, None]
  n2c = norm2[:, None]

  prop = _sc_prop(nbw0, nbw1)
  raw1 = prop(feats_p * n2c, srcp, dstp)
  in2 = (raw1[0] + raw1[1]) * s11
  raw2 = prop(in2, srcp, dstp)
  r2 = raw2[0] + raw2[1]
  y = feats_p + r2 * n2c
  raw3 = prop(r2 * s12, srcp, dstp)
  r3 = raw3[0] + raw3[1]
  y = y + r3 * n2c
  raw4 = prop(r3 * s12, srcp, dstp)
  r4 = raw4[0] + raw4[1]
  y = (y + r4 * n2c) * 0.25

  z, el, er, cm = _tc_gat(y, W, attn_l, attn_r)
  c16 = jnp.full((16,), cm[0, 0] + cm[1, 0], jnp.float32)
  nump, denp = _sc_gat(nbw0g, nbw1g)(
      z, el.reshape(NPAD), er.reshape(NPAD), c16, srcp, dstp)
  num = nump[0] + nump[1]
  den = (denp[0] + denp[1])[:, None]
  logp = _tc_final(num, den, bias)
  return logp[:n]
